# Initial kernel scaffold; baseline (speedup 1.0000x reference)
import jax
import jax.numpy as jnp
from jax.experimental import pallas as pl


def kernel(x, edge_index, edge_vec, Wl0, Wself0, fc1_0, fc2_0, Wl1, Wself1, fc1_1, fc2_1, Wl2, Wself2, fc1_2, fc2_2):
    raise NotImplementedError("write your pallas kernel here")


# speedup vs baseline: 1.5692x; 1.5692x over previous
import functools

import jax
import jax.numpy as jnp
import numpy as np
from jax import lax
from jax.experimental import pallas as pl
from jax.experimental.pallas import tpu as pltpu
from jax.experimental.pallas import tpu_sc as plsc

N_NODES = 10000
N_EDGES = 160000
NPAD = 10240
EPAD = 163840
CH = 128
EPW = EPAD // 32
NCH = EPW // CH
ZR = NPAD // 16

_F32 = jnp.float32
_STEP = 3.5 / 11.0
_EMB_SCALE = 1.14136 * float(np.exp(2.0)) * float(np.sqrt(10.0))


def _coef_body(ev_ref, f1t0, f2t0, f1t1, f2t1, f1t2, f2t2, c_ref):
    ev = ev_ref[...]
    x = ev[0:1, :]
    y = ev[1:2, :]
    z = ev[2:3, :]
    r2 = x * x + y * y + z * z
    r = jnp.sqrt(r2 + 1e-12)
    inv = 1.0 / r
    ux, uy, uz = x * inv, y * inv, z * inv
    s1 = 1.7320508075688772 * (ux + uy + uz)
    s2 = (3.872983346207417 * (ux * uy + uy * uz + ux * uz)
          + 1.118033988749895 * (3.0 * uz * uz - 1.0)
          + 1.9364916731037085 * (ux * ux - uy * uy))
    s_all = jnp.concatenate([jnp.ones_like(s1), s1, s2], axis=0)

    vals = (lax.broadcasted_iota(jnp.int32, (10, 1), 0).astype(_F32) + 1.0) * _STEP
    diff = (r - vals) * (1.0 / _STEP)

    def _sus(t):
        return jnp.where(t > 0.0, jnp.exp(-1.0 / jnp.clip(t, 1e-8, None)), 0.0)

    emb = _EMB_SCALE * _sus(diff + 1.0) * _sus(1.0 - diff)

    rows = []
    for f1t, f2t in ((f1t0, f2t0), (f1t1, f2t1), (f1t2, f2t2)):
        hid = jnp.dot(f1t[...], emb, preferred_element_type=_F32)
        hid = hid * (1.0 / (1.0 + jnp.exp(-hid)))
        w = jnp.dot(f2t[...], hid, preferred_element_type=_F32)
        rows.append(w * s_all)
    c = jnp.concatenate(rows, axis=0)
    c_ref[...] = jnp.concatenate([c, jnp.zeros_like(c[0:7, :])], axis=0)


def _coef_call(evt, f1ts, f2ts):
    be = 8192
    grid = (EPAD // be,)
    full = pl.BlockSpec((None, None), lambda i: (0, 0))
    specs = [pl.BlockSpec((3, be), lambda i: (0, i))]
    for _ in range(3):
        specs.append(pl.BlockSpec((100, 10), lambda i: (0, 0)))
        specs.append(pl.BlockSpec((3, 100), lambda i: (0, 0)))
    args = [evt]
    for f1t, f2t in zip(f1ts, f2ts):
        args += [f1t, f2t]
    return pl.pallas_call(
        _coef_body,
        grid=grid,
        in_specs=specs,
        out_specs=pl.BlockSpec((16, be), lambda i: (0, i)),
        out_shape=jax.ShapeDtypeStruct((16, EPAD), _F32),
    )(*args)


def _mm0_body(x_ref, w_ref, y_ref, s_ref):
    t = jnp.dot(x_ref[...], w_ref[...], preferred_element_type=_F32)
    y_ref[...] = t[:, :192]
    s_ref[...] = t[:, 192:]


def _mm0_call(x, wcat):
    br = 2000
    return pl.pallas_call(
        _mm0_body,
        grid=(N_NODES // br,),
        in_specs=[pl.BlockSpec((br, 256), lambda i: (i, 0)),
                  pl.BlockSpec((256, 256), lambda i: (0, 0))],
        out_specs=[pl.BlockSpec((br, 192), lambda i: (i, 0)),
                   pl.BlockSpec((br, 64), lambda i: (i, 0))],
        out_shape=[jax.ShapeDtypeStruct((N_NODES, 192), _F32),
                   jax.ShapeDtypeStruct((N_NODES, 64), _F32)],
    )(x, wcat)


def _mmc_body(s_ref, acc_ref, w_ref, y_ref, sn_ref):
    pre = s_ref[...] + 0.25 * (acc_ref[0] + acc_ref[1])
    h = pre * (1.0 / (1.0 + jnp.exp(-pre)))
    t = jnp.dot(h, w_ref[...], preferred_element_type=_F32)
    y_ref[...] = t[:, :192]
    sn_ref[...] = t[:, 192:]


def _mmc_call(s_prev, acc, wcat):
    br = 2000
    return pl.pallas_call(
        _mmc_body,
        grid=(N_NODES // br,),
        in_specs=[pl.BlockSpec((br, 64), lambda i: (i, 0)),
                  pl.BlockSpec((2, br, 64), lambda i: (0, i, 0)),
                  pl.BlockSpec((64, 256), lambda i: (0, 0))],
        out_specs=[pl.BlockSpec((br, 192), lambda i: (i, 0)),
                   pl.BlockSpec((br, 64), lambda i: (i, 0))],
        out_shape=[jax.ShapeDtypeStruct((N_NODES, 192), _F32),
                   jax.ShapeDtypeStruct((N_NODES, 64), _F32)],
    )(s_prev, acc, wcat)


def _comb2_body(s_ref, acc_ref, h_ref):
    pre = s_ref[...] + 0.25 * (acc_ref[0] + acc_ref[1])
    h_ref[...] = pre * (1.0 / (1.0 + jnp.exp(-pre)))


def _comb2_call(s_prev, acc):
    br = 2000
    return pl.pallas_call(
        _comb2_body,
        grid=(N_NODES // br,),
        in_specs=[pl.BlockSpec((br, 64), lambda i: (i, 0)),
                  pl.BlockSpec((2, br, 64), lambda i: (0, i, 0))],
        out_specs=pl.BlockSpec((br, 64), lambda i: (i, 0)),
        out_shape=jax.ShapeDtypeStruct((N_NODES, 64), _F32),
    )(s_prev, acc)


def _final_body(h_ref, a_ref, ws_ref, w2_ref, o_ref):
    a = jnp.concatenate([a_ref[0], a_ref[1][:, :64]], axis=1)
    o_ref[...] = (jnp.dot(h_ref[...], ws_ref[...], preferred_element_type=_F32)
                  + 0.25 * jnp.dot(a, w2_ref[...], preferred_element_type=_F32))


def _final_call(h2, acc, wself2, w2stack):
    br = 2000
    return pl.pallas_call(
        _final_body,
        grid=(N_NODES // br,),
        in_specs=[pl.BlockSpec((br, 64), lambda i: (i, 0)),
                  pl.BlockSpec((2, br, 128), lambda i: (0, i, 0)),
                  pl.BlockSpec((64, 256), lambda i: (0, 0)),
                  pl.BlockSpec((192, 256), lambda i: (0, 0))],
        out_specs=pl.BlockSpec((br, 256), lambda i: (i, 0)),
        out_shape=jax.ShapeDtypeStruct((N_NODES, 256), _F32),
    )(h2, acc, wself2, w2stack)


def _make_scgs(mode):
    gw = 192 if mode == 0 else 64
    sw = 64 if mode == 0 else 128
    nch = NCH if mode == 0 else EPAD // 16 // CH
    mesh = plsc.VectorSubcoreMesh(core_axis_name="c", subcore_axis_name="s",
                                  num_cores=2, num_subcores=16)

    @functools.partial(
        pl.kernel,
        out_type=jax.ShapeDtypeStruct((2, NPAD, sw), _F32),
        mesh=mesh,
        compiler_params=pltpu.CompilerParams(use_tc_tiling_on_sc=False),
        scratch_types=[
            pltpu.VMEM((CH,), jnp.int32),
            pltpu.VMEM((CH,), jnp.int32),
            pltpu.VMEM((CH + 16,), _F32),
            pltpu.VMEM((CH + 16,), _F32),
            pltpu.VMEM((CH + 16,), _F32),
            pltpu.VMEM((CH, gw), _F32),
            pltpu.VMEM((CH, sw), _F32),
            pltpu.VMEM_SHARED((NPAD, sw), _F32),
            pltpu.SemaphoreType.DMA,
        ],
    )
    def scgs(y_hbm, si_hbm, di_hbm, c0_hbm, c1_hbm, c2_hbm, z_hbm, acc_hbm,
             si_v, di_v, c0_v, c1_v, c2_v, rows_v, msg_v, acc_sh, sem):
        cid = lax.axis_index("c")
        sid = lax.axis_index("s")

        pltpu.sync_copy(z_hbm.at[pl.ds(sid * ZR, ZR)],
                        acc_sh.at[pl.ds(sid * ZR, ZR)])
        plsc.subcore_barrier()

        if mode == 0:
            base0 = (sid * 2 + cid) * EPW
        else:
            base0 = sid * (EPAD // 16)

        def chunk_body(i, carry):
            base = base0 + i * CH
            pltpu.sync_copy(si_hbm.at[pl.ds(base, CH)], si_v)
            pltpu.sync_copy(di_hbm.at[pl.ds(base, CH)], di_v)
            pltpu.sync_copy(c0_hbm.at[pl.ds(base, CH)], c0_v.at[pl.ds(0, CH)])
            pltpu.sync_copy(c1_hbm.at[pl.ds(base, CH)], c1_v.at[pl.ds(0, CH)])
            pltpu.sync_copy(c2_hbm.at[pl.ds(base, CH)], c2_v.at[pl.ds(0, CH)])
            pltpu.async_copy(y_hbm.at[si_v], rows_v, sem).wait()

            def edge_body(e, c):
                c0 = c0_v[pl.ds(e, 16)][0]
                c1 = c1_v[pl.ds(e, 16)][0]
                c2 = c2_v[pl.ds(e, 16)][0]
                if mode == 0:
                    for j in range(4):
                        msg_v[e, pl.ds(j * 16, 16)] = (
                            c0 * rows_v[e, pl.ds(j * 16, 16)]
                            + c1 * rows_v[e, pl.ds(64 + j * 16, 16)]
                            + c2 * rows_v[e, pl.ds(128 + j * 16, 16)])
                else:
                    ca = jnp.where(cid == 0, c0, c2)
                    cb = jnp.where(cid == 0, c1, 0.0)
                    for j in range(4):
                        hv = rows_v[e, pl.ds(j * 16, 16)]
                        msg_v[e, pl.ds(j * 16, 16)] = ca * hv
                        msg_v[e, pl.ds(64 + j * 16, 16)] = cb * hv
                return c

            lax.fori_loop(0, CH, edge_body, 0)
            pltpu.sync_copy(msg_v, acc_sh.at[di_v], add=True)
            return carry

        lax.fori_loop(0, nch, chunk_body, 0)
        plsc.subcore_barrier()
        pltpu.sync_copy(acc_sh.at[pl.ds(sid * ZR, ZR)],
                        acc_hbm.at[cid, pl.ds(sid * ZR, ZR)])

    return scgs


@functools.lru_cache(maxsize=None)
def _get_scgs(mode):
    return _make_scgs(mode)


def _scgs_wide(*args):
    return _get_scgs(0)(*args)


def _scgs_outer(*args):
    return _get_scgs(1)(*args)


def kernel(x, edge_index, edge_vec, Wl0, Wself0, fc1_0, fc2_0,
           Wl1, Wself1, fc1_1, fc2_1, Wl2, Wself2, fc1_2, fc2_2):
    pad_e = EPAD - N_EDGES
    src = jnp.concatenate([edge_index[0], jnp.zeros((pad_e,), jnp.int32)])
    dst = jnp.concatenate([edge_index[1], jnp.zeros((pad_e,), jnp.int32)])
    evt = jnp.pad(edge_vec.T, ((0, 0), (0, pad_e)))

    wcat0 = jnp.concatenate([Wl0[0], Wl0[1], Wl0[2], Wself0], axis=1)
    wcat1 = jnp.concatenate([Wl1[0], Wl1[1], Wl1[2], Wself1], axis=1)
    w2stack = jnp.concatenate([Wl2[0], Wl2[1], Wl2[2]], axis=0)
    zeros64 = jnp.zeros((NPAD, 64), _F32)
    zeros128 = jnp.zeros((NPAD, 128), _F32)

    c_all = _coef_call(evt,
                       (fc1_0.T, fc1_1.T, fc1_2.T),
                       (fc2_0.T, fc2_1.T, fc2_2.T))

    y0, s0 = _mm0_call(x, wcat0)
    acc0 = _scgs_wide(y0, src, dst, c_all[0], c_all[1], c_all[2], zeros64)
    y1, s1 = _mmc_call(s0, acc0, wcat1)
    acc1 = _scgs_wide(y1, src, dst, c_all[3], c_all[4], c_all[5], zeros64)
    h2 = _comb2_call(s1, acc1)
    a2 = _scgs_outer(h2, src, dst, c_all[6], c_all[7], c_all[8], zeros128)
    return _final_call(h2, a2, Wself2, w2stack)


# speedup vs baseline: 2.6099x; 1.6632x over previous
import functools

import jax
import jax.numpy as jnp
import numpy as np
from jax import lax
from jax.experimental import pallas as pl
from jax.experimental.pallas import tpu as pltpu
from jax.experimental.pallas import tpu_sc as plsc

N_NODES = 10000
N_EDGES = 160000
NPAD = 10240
EPAD = 163840
CH = 128
EPW = EPAD // 32
NCH = EPW // CH
ZR = NPAD // 16

_F32 = jnp.float32
_STEP = 3.5 / 11.0
_EMB_SCALE = 1.14136 * float(np.exp(2.0)) * float(np.sqrt(10.0))


def _coef_body(ev_ref, f1t0, f2t0, f1t1, f2t1, f1t2, f2t2, c_ref):
    ev = ev_ref[...]
    x = ev[0:1, :]
    y = ev[1:2, :]
    z = ev[2:3, :]
    r2 = x * x + y * y + z * z
    r = jnp.sqrt(r2 + 1e-12)
    inv = 1.0 / r
    ux, uy, uz = x * inv, y * inv, z * inv
    s1 = 1.7320508075688772 * (ux + uy + uz)
    s2 = (3.872983346207417 * (ux * uy + uy * uz + ux * uz)
          + 1.118033988749895 * (3.0 * uz * uz - 1.0)
          + 1.9364916731037085 * (ux * ux - uy * uy))
    s_all = jnp.concatenate([jnp.ones_like(s1), s1, s2], axis=0)

    vals = (lax.broadcasted_iota(jnp.int32, (10, 1), 0).astype(_F32) + 1.0) * _STEP
    diff = (r - vals) * (1.0 / _STEP)

    def _sus(t):
        return jnp.where(t > 0.0, jnp.exp(-1.0 / jnp.clip(t, 1e-8, None)), 0.0)

    emb = _EMB_SCALE * _sus(diff + 1.0) * _sus(1.0 - diff)

    rows = []
    for f1t, f2t in ((f1t0, f2t0), (f1t1, f2t1), (f1t2, f2t2)):
        hid = jnp.dot(f1t[...], emb, preferred_element_type=_F32)
        hid = hid * (1.0 / (1.0 + jnp.exp(-hid)))
        w = jnp.dot(f2t[...], hid, preferred_element_type=_F32)
        rows.append(w * s_all)
    c = jnp.concatenate(rows, axis=0)
    c_ref[...] = jnp.concatenate([c, jnp.zeros_like(c[0:7, :])], axis=0)


def _coef_call(evt, f1ts, f2ts):
    be = 8192
    grid = (EPAD // be,)
    full = pl.BlockSpec((None, None), lambda i: (0, 0))
    specs = [pl.BlockSpec((3, be), lambda i: (0, i))]
    for _ in range(3):
        specs.append(pl.BlockSpec((100, 10), lambda i: (0, 0)))
        specs.append(pl.BlockSpec((3, 100), lambda i: (0, 0)))
    args = [evt]
    for f1t, f2t in zip(f1ts, f2ts):
        args += [f1t, f2t]
    return pl.pallas_call(
        _coef_body,
        grid=grid,
        in_specs=specs,
        out_specs=pl.BlockSpec((16, be), lambda i: (0, i)),
        out_shape=jax.ShapeDtypeStruct((16, EPAD), _F32),
    )(*args)


def _mm0_body(x_ref, w_ref, y_ref, s_ref):
    t = jnp.dot(x_ref[...], w_ref[...], preferred_element_type=_F32)
    y_ref[...] = t[:, :192]
    s_ref[...] = t[:, 192:]


def _mm0_call(x, wcat):
    br = 2000
    return pl.pallas_call(
        _mm0_body,
        grid=(N_NODES // br,),
        in_specs=[pl.BlockSpec((br, 256), lambda i: (i, 0)),
                  pl.BlockSpec((256, 256), lambda i: (0, 0))],
        out_specs=[pl.BlockSpec((br, 192), lambda i: (i, 0)),
                   pl.BlockSpec((br, 64), lambda i: (i, 0))],
        out_shape=[jax.ShapeDtypeStruct((N_NODES, 192), _F32),
                   jax.ShapeDtypeStruct((N_NODES, 64), _F32)],
    )(x, wcat)


def _mmc_body(s_ref, acc_ref, w_ref, y_ref, sn_ref):
    pre = s_ref[...] + 0.25 * (acc_ref[0] + acc_ref[1])
    h = pre * (1.0 / (1.0 + jnp.exp(-pre)))
    t = jnp.dot(h, w_ref[...], preferred_element_type=_F32)
    y_ref[...] = t[:, :192]
    sn_ref[...] = t[:, 192:]


def _mmc_call(s_prev, acc, wcat):
    br = 2000
    return pl.pallas_call(
        _mmc_body,
        grid=(N_NODES // br,),
        in_specs=[pl.BlockSpec((br, 64), lambda i: (i, 0)),
                  pl.BlockSpec((2, br, 64), lambda i: (0, i, 0)),
                  pl.BlockSpec((64, 256), lambda i: (0, 0))],
        out_specs=[pl.BlockSpec((br, 192), lambda i: (i, 0)),
                   pl.BlockSpec((br, 64), lambda i: (i, 0))],
        out_shape=[jax.ShapeDtypeStruct((N_NODES, 192), _F32),
                   jax.ShapeDtypeStruct((N_NODES, 64), _F32)],
    )(s_prev, acc, wcat)


def _comb2_body(s_ref, acc_ref, h_ref):
    pre = s_ref[...] + 0.25 * (acc_ref[0] + acc_ref[1])
    h_ref[...] = pre * (1.0 / (1.0 + jnp.exp(-pre)))


def _comb2_call(s_prev, acc):
    br = 2000
    return pl.pallas_call(
        _comb2_body,
        grid=(N_NODES // br,),
        in_specs=[pl.BlockSpec((br, 64), lambda i: (i, 0)),
                  pl.BlockSpec((2, br, 64), lambda i: (0, i, 0))],
        out_specs=pl.BlockSpec((br, 64), lambda i: (i, 0)),
        out_shape=jax.ShapeDtypeStruct((N_NODES, 64), _F32),
    )(s_prev, acc)


def _final_body(h_ref, a_ref, ws_ref, w2_ref, o_ref):
    a = jnp.concatenate([a_ref[0], a_ref[1]], axis=1)
    o_ref[...] = (jnp.dot(h_ref[...], ws_ref[...], preferred_element_type=_F32)
                  + 0.25 * jnp.dot(a, w2_ref[...], preferred_element_type=_F32))


def _final_call(h2, acc, wself2, w2stack):
    br = 2000
    return pl.pallas_call(
        _final_body,
        grid=(N_NODES // br,),
        in_specs=[pl.BlockSpec((br, 64), lambda i: (i, 0)),
                  pl.BlockSpec((2, br, 96), lambda i: (0, i, 0)),
                  pl.BlockSpec((64, 256), lambda i: (0, 0)),
                  pl.BlockSpec((192, 256), lambda i: (0, 0))],
        out_specs=pl.BlockSpec((br, 256), lambda i: (i, 0)),
        out_shape=jax.ShapeDtypeStruct((N_NODES, 256), _F32),
    )(h2, acc, wself2, w2stack)


def _make_scgs(mode):
    gw = 192 if mode == 0 else 64
    sw = 64 if mode == 0 else 96
    nch = NCH if mode == 0 else EPAD // 16 // CH
    mesh = plsc.VectorSubcoreMesh(core_axis_name="c", subcore_axis_name="s",
                                  num_cores=2, num_subcores=16)

    @functools.partial(
        pl.kernel,
        out_type=jax.ShapeDtypeStruct((2, NPAD, sw), _F32),
        mesh=mesh,
        compiler_params=pltpu.CompilerParams(use_tc_tiling_on_sc=False),
        scratch_types=[
            pltpu.VMEM((5, CH), jnp.int32),
            pltpu.VMEM((5, CH), jnp.int32),
            pltpu.VMEM((5, CH), jnp.int32),
            pltpu.VMEM((5, CH), jnp.int32),
            pltpu.VMEM((CH, gw), _F32),
            pltpu.VMEM((CH, gw), _F32),
            pltpu.VMEM((CH, sw), _F32),
            pltpu.VMEM((CH, sw), _F32),
            pltpu.VMEM_SHARED((NPAD, sw), _F32),
            pltpu.SemaphoreType.DMA,
            pltpu.SemaphoreType.DMA,
            pltpu.SemaphoreType.DMA,
            pltpu.SemaphoreType.DMA,
            pltpu.SemaphoreType.DMA,
            pltpu.SemaphoreType.DMA,
            pltpu.SemaphoreType.DMA,
            pltpu.SemaphoreType.DMA,
        ],
    )
    def scgs(y_hbm, ed_hbm, z_hbm, acc_hbm,
             ed0, ed1, ed2, ed3, rw0, rw1, mg0, mg1, acc_sh,
             sg0, sg1, se0, se1, se2, se3, ss0, ss1):
        cid = lax.axis_index("c")
        sid = lax.axis_index("s")
        eds = (ed0, ed1, ed2, ed3)
        rows = (rw0, rw1)
        msgs = (mg0, mg1)
        sgs = (sg0, sg1)
        ses = (se0, se1, se2, se3)
        sss = (ss0, ss1)

        pltpu.sync_copy(z_hbm.at[pl.ds(sid * ZR, ZR)],
                        acc_sh.at[pl.ds(sid * ZR, ZR)])
        plsc.subcore_barrier()

        if mode == 0:
            ci0 = (sid * 2 + cid) * nch
        else:
            ci0 = sid * nch
            mf = jnp.where(cid == 0, 1.0, 0.0)
            o0 = jnp.where(cid == 0, 0, 32)
            o1 = jnp.where(cid == 0, 16, 48)
            o2 = jnp.where(cid == 0, 32, 0)
            o3 = jnp.where(cid == 0, 48, 16)

        def ed_src(c):
            return ed_hbm.at[pl.ds((ci0 + c) * 5, 5)]

        pltpu.sync_copy(ed_src(0), ed0)
        pltpu.async_copy(y_hbm.at[ed0.at[0]], rw0, sg0)
        pltpu.async_copy(ed_src(1), ed1, se1)

        def compute_chunk(ed_v, rows_v, msg_v):
            def group_body(g, carry):
                c0g = lax.bitcast_convert_type(ed_v[2, pl.ds(g * 16, 16)], _F32)
                c1g = lax.bitcast_convert_type(ed_v[3, pl.ds(g * 16, 16)], _F32)
                c2g = lax.bitcast_convert_type(ed_v[4, pl.ds(g * 16, 16)], _F32)
                if mode != 0:
                    pa = c0g * mf + c1g * (1.0 - mf)
                    pb = c0g * mf + c2g * (1.0 - mf)
                    pc = c1g * mf + c2g * (1.0 - mf)
                for t in range(16):
                    e = g * 16 + t
                    if mode == 0:
                        c0 = c0g[t]
                        c1 = c1g[t]
                        c2 = c2g[t]
                        for j in range(4):
                            msg_v[e, pl.ds(j * 16, 16)] = (
                                c0 * rows_v[e, pl.ds(j * 16, 16)]
                                + c1 * rows_v[e, pl.ds(64 + j * 16, 16)]
                                + c2 * rows_v[e, pl.ds(128 + j * 16, 16)])
                    else:
                        for p, (cg, off) in enumerate(
                                ((pa, o0), (pa, o1), (pb, o2),
                                 (pb, o3), (pc, o0), (pc, o1))):
                            msg_v[e, pl.ds(p * 16, 16)] = (
                                cg[t] * rows_v[e, pl.ds(off, 16)])
                return carry

            lax.fori_loop(0, CH // 16, group_body, 0)

        def step(s, carry):
            for b in range(4):
                c = s * 4 + b
                gb = b % 2
                nb = (b + 1) % 2
                pltpu.make_async_copy(
                    y_hbm.at[eds[b].at[0]], rows[gb], sgs[gb]).wait()

                @pl.when(c + 1 < nch)
                def _():
                    pltpu.make_async_copy(
                        ed_src(c + 1), eds[(b + 1) % 4],
                        ses[(b + 1) % 4]).wait()
                    pltpu.async_copy(
                        y_hbm.at[eds[(b + 1) % 4].at[0]], rows[nb], sgs[nb])

                @pl.when(c >= 2)
                def _():
                    pltpu.make_async_copy(
                        msgs[gb], acc_sh.at[eds[(b + 2) % 4].at[1]],
                        sss[gb]).wait()

                @pl.when(c + 2 < nch)
                def _():
                    pltpu.async_copy(ed_src(c + 2), eds[(b + 2) % 4],
                                     ses[(b + 2) % 4])

                compute_chunk(eds[b], rows[gb], msgs[gb])
                pltpu.async_copy(msgs[gb], acc_sh.at[eds[b].at[1]],
                                 sss[gb], add=True)
            return carry

        lax.fori_loop(0, nch // 4, step, 0)
        pltpu.make_async_copy(msgs[0], acc_sh.at[eds[2].at[1]], sss[0]).wait()
        pltpu.make_async_copy(msgs[1], acc_sh.at[eds[3].at[1]], sss[1]).wait()
        plsc.subcore_barrier()
        pltpu.sync_copy(acc_sh.at[pl.ds(sid * ZR, ZR)],
                        acc_hbm.at[cid, pl.ds(sid * ZR, ZR)])

    return scgs


@functools.lru_cache(maxsize=None)
def _get_scgs(mode):
    return _make_scgs(mode)


def _scgs_wide(*args):
    return _get_scgs(0)(*args)


def _scgs_outer(*args):
    return _get_scgs(1)(*args)


def _pack_ed(src_r, dst_r, c0, c1, c2):
    cb = [lax.bitcast_convert_type(c.reshape(EPAD // CH, CH), jnp.int32)
          for c in (c0, c1, c2)]
    return jnp.stack([src_r, dst_r, *cb], axis=1).reshape(EPAD // CH * 5, CH)


def kernel(x, edge_index, edge_vec, Wl0, Wself0, fc1_0, fc2_0,
           Wl1, Wself1, fc1_1, fc2_1, Wl2, Wself2, fc1_2, fc2_2):
    pad_e = EPAD - N_EDGES
    src = jnp.concatenate([edge_index[0], jnp.zeros((pad_e,), jnp.int32)])
    dst = jnp.concatenate([edge_index[1], jnp.zeros((pad_e,), jnp.int32)])
    evt = jnp.pad(edge_vec.T, ((0, 0), (0, pad_e)))

    wcat0 = jnp.concatenate([Wl0[0], Wl0[1], Wl0[2], Wself0], axis=1)
    wcat1 = jnp.concatenate([Wl1[0], Wl1[1], Wl1[2], Wself1], axis=1)
    w2stack = jnp.concatenate([Wl2[0], Wl2[1], Wl2[2]], axis=0)
    zeros64 = jnp.zeros((NPAD, 64), _F32)
    zeros96 = jnp.zeros((NPAD, 96), _F32)

    c_all = _coef_call(evt,
                       (fc1_0.T, fc1_1.T, fc1_2.T),
                       (fc2_0.T, fc2_1.T, fc2_2.T))

    src_r = src.reshape(EPAD // CH, CH)
    dst_r = dst.reshape(EPAD // CH, CH)
    ed_l0 = _pack_ed(src_r, dst_r, c_all[0], c_all[1], c_all[2])
    ed_l1 = _pack_ed(src_r, dst_r, c_all[3], c_all[4], c_all[5])
    ed_l2 = _pack_ed(src_r, dst_r, c_all[6], c_all[7], c_all[8])

    y0, s0 = _mm0_call(x, wcat0)
    acc0 = _scgs_wide(y0, ed_l0, zeros64)
    y1, s1 = _mmc_call(s0, acc0, wcat1)
    acc1 = _scgs_wide(y1, ed_l1, zeros64)
    h2 = _comb2_call(s1, acc1)
    a2 = _scgs_outer(h2, ed_l2, zeros96)
    return _final_call(h2, a2, Wself2, w2stack)


# speedup vs baseline: 2.7239x; 1.0437x over previous
import functools

import jax
import jax.numpy as jnp
import numpy as np
from jax import lax
from jax.experimental import pallas as pl
from jax.experimental.pallas import tpu as pltpu
from jax.experimental.pallas import tpu_sc as plsc

N_NODES = 10000
N_EDGES = 160000
NPAD = 10240
EPAD = 163840
CH = 128
EPW = EPAD // 32
NCH = EPW // CH
ZR = NPAD // 16

_F32 = jnp.float32
_STEP = 3.5 / 11.0
_EMB_SCALE = 1.14136 * float(np.exp(2.0)) * float(np.sqrt(10.0))


def _coef_body(ev_ref, f1t0, f2t0, f1t1, f2t1, f1t2, f2t2, c_ref):
    ev = ev_ref[...]
    x = ev[0:1, :]
    y = ev[1:2, :]
    z = ev[2:3, :]
    r2 = x * x + y * y + z * z
    r = jnp.sqrt(r2 + 1e-12)
    inv = 1.0 / r
    ux, uy, uz = x * inv, y * inv, z * inv
    s1 = 1.7320508075688772 * (ux + uy + uz)
    s2 = (3.872983346207417 * (ux * uy + uy * uz + ux * uz)
          + 1.118033988749895 * (3.0 * uz * uz - 1.0)
          + 1.9364916731037085 * (ux * ux - uy * uy))
    s_all = jnp.concatenate([jnp.ones_like(s1), s1, s2], axis=0)

    vals = (lax.broadcasted_iota(jnp.int32, (10, 1), 0).astype(_F32) + 1.0) * _STEP
    diff = (r - vals) * (1.0 / _STEP)

    def _sus(t):
        return jnp.where(t > 0.0, jnp.exp(-1.0 / jnp.clip(t, 1e-8, None)), 0.0)

    emb = _EMB_SCALE * _sus(diff + 1.0) * _sus(1.0 - diff)

    rows = []
    for f1t, f2t in ((f1t0, f2t0), (f1t1, f2t1), (f1t2, f2t2)):
        hid = jnp.dot(f1t[...], emb, preferred_element_type=_F32)
        hid = hid * (1.0 / (1.0 + jnp.exp(-hid)))
        w = jnp.dot(f2t[...], hid, preferred_element_type=_F32)
        rows.append(w * s_all)
    c = jnp.concatenate(rows, axis=0)
    c_ref[...] = jnp.concatenate([c, jnp.zeros_like(c[0:7, :])], axis=0)


def _coef_call(evt, f1ts, f2ts):
    be = 8192
    grid = (EPAD // be,)
    full = pl.BlockSpec((None, None), lambda i: (0, 0))
    specs = [pl.BlockSpec((3, be), lambda i: (0, i))]
    for _ in range(3):
        specs.append(pl.BlockSpec((100, 10), lambda i: (0, 0)))
        specs.append(pl.BlockSpec((3, 100), lambda i: (0, 0)))
    args = [evt]
    for f1t, f2t in zip(f1ts, f2ts):
        args += [f1t, f2t]
    return pl.pallas_call(
        _coef_body,
        grid=grid,
        in_specs=specs,
        out_specs=pl.BlockSpec((16, be), lambda i: (0, i)),
        out_shape=jax.ShapeDtypeStruct((16, EPAD), _F32),
    )(*args)


def _mm0_body(x_ref, w_ref, y_ref, s_ref):
    t = jnp.dot(x_ref[...], w_ref[...], preferred_element_type=_F32)
    y_ref[...] = t[:, :192]
    s_ref[...] = t[:, 192:]


def _mm0_call(x, wcat):
    br = 2000
    return pl.pallas_call(
        _mm0_body,
        grid=(N_NODES // br,),
        in_specs=[pl.BlockSpec((br, 256), lambda i: (i, 0)),
                  pl.BlockSpec((256, 256), lambda i: (0, 0))],
        out_specs=[pl.BlockSpec((br, 192), lambda i: (i, 0)),
                   pl.BlockSpec((br, 64), lambda i: (i, 0))],
        out_shape=[jax.ShapeDtypeStruct((N_NODES, 192), _F32),
                   jax.ShapeDtypeStruct((N_NODES, 64), _F32)],
    )(x, wcat)


def _mmc_body(s_ref, acc_ref, w_ref, y_ref, sn_ref):
    pre = s_ref[...] + 0.25 * (acc_ref[0] + acc_ref[1])
    h = pre * (1.0 / (1.0 + jnp.exp(-pre)))
    t = jnp.dot(h, w_ref[...], preferred_element_type=_F32)
    y_ref[...] = t[:, :192]
    sn_ref[...] = t[:, 192:]


def _mmc_call(s_prev, acc, wcat):
    br = 2000
    return pl.pallas_call(
        _mmc_body,
        grid=(N_NODES // br,),
        in_specs=[pl.BlockSpec((br, 64), lambda i: (i, 0)),
                  pl.BlockSpec((2, br, 64), lambda i: (0, i, 0)),
                  pl.BlockSpec((64, 256), lambda i: (0, 0))],
        out_specs=[pl.BlockSpec((br, 192), lambda i: (i, 0)),
                   pl.BlockSpec((br, 64), lambda i: (i, 0))],
        out_shape=[jax.ShapeDtypeStruct((N_NODES, 192), _F32),
                   jax.ShapeDtypeStruct((N_NODES, 64), _F32)],
    )(s_prev, acc, wcat)


def _comb2_body(s_ref, acc_ref, h_ref):
    pre = s_ref[...] + 0.25 * (acc_ref[0] + acc_ref[1])
    h_ref[...] = pre * (1.0 / (1.0 + jnp.exp(-pre)))


def _comb2_call(s_prev, acc):
    br = 2000
    return pl.pallas_call(
        _comb2_body,
        grid=(N_NODES // br,),
        in_specs=[pl.BlockSpec((br, 64), lambda i: (i, 0)),
                  pl.BlockSpec((2, br, 64), lambda i: (0, i, 0))],
        out_specs=pl.BlockSpec((br, 64), lambda i: (i, 0)),
        out_shape=jax.ShapeDtypeStruct((N_NODES, 64), _F32),
    )(s_prev, acc)


def _final_body(h_ref, a_ref, ws_ref, w2_ref, o_ref):
    a = jnp.concatenate([a_ref[0], a_ref[1]], axis=1)
    o_ref[...] = (jnp.dot(h_ref[...], ws_ref[...], preferred_element_type=_F32)
                  + 0.25 * jnp.dot(a, w2_ref[...], preferred_element_type=_F32))


def _final_call(h2, acc, wself2, w2stack):
    br = 2000
    return pl.pallas_call(
        _final_body,
        grid=(N_NODES // br,),
        in_specs=[pl.BlockSpec((br, 64), lambda i: (i, 0)),
                  pl.BlockSpec((2, br, 96), lambda i: (0, i, 0)),
                  pl.BlockSpec((64, 256), lambda i: (0, 0)),
                  pl.BlockSpec((192, 256), lambda i: (0, 0))],
        out_specs=pl.BlockSpec((br, 256), lambda i: (i, 0)),
        out_shape=jax.ShapeDtypeStruct((N_NODES, 256), _F32),
    )(h2, acc, wself2, w2stack)


def _make_scgs(mode):
    gw = 192 if mode == 0 else 64
    sw = 64 if mode == 0 else 96
    nch = NCH if mode == 0 else EPAD // 16 // CH
    mesh = plsc.VectorSubcoreMesh(core_axis_name="c", subcore_axis_name="s",
                                  num_cores=2, num_subcores=16)

    @functools.partial(
        pl.kernel,
        out_type=jax.ShapeDtypeStruct((2, NPAD, sw), _F32),
        mesh=mesh,
        compiler_params=pltpu.CompilerParams(use_tc_tiling_on_sc=False),
        scratch_types=[
            pltpu.VMEM((5, CH), jnp.int32),
            pltpu.VMEM((5, CH), jnp.int32),
            pltpu.VMEM((5, CH), jnp.int32),
            pltpu.VMEM((5, CH), jnp.int32),
            pltpu.VMEM((CH, gw), _F32),
            pltpu.VMEM((CH, gw), _F32),
            pltpu.VMEM((CH, sw), _F32),
            pltpu.VMEM((CH, sw), _F32),
            pltpu.VMEM_SHARED((NPAD, sw), _F32),
            pltpu.SemaphoreType.DMA,
            pltpu.SemaphoreType.DMA,
            pltpu.SemaphoreType.DMA,
            pltpu.SemaphoreType.DMA,
            pltpu.SemaphoreType.DMA,
            pltpu.SemaphoreType.DMA,
            pltpu.SemaphoreType.DMA,
            pltpu.SemaphoreType.DMA,
        ],
    )
    def scgs(y_hbm, ed_hbm, z_hbm, acc_hbm,
             ed0, ed1, ed2, ed3, rw0, rw1, mg0, mg1, acc_sh,
             sg0, sg1, se0, se1, se2, se3, ss0, ss1):
        cid = lax.axis_index("c")
        sid = lax.axis_index("s")
        eds = (ed0, ed1, ed2, ed3)
        rows = (rw0, rw1)
        msgs = (mg0, mg1)
        sgs = (sg0, sg1)
        ses = (se0, se1, se2, se3)
        sss = (ss0, ss1)

        pltpu.sync_copy(z_hbm.at[pl.ds(sid * ZR, ZR)],
                        acc_sh.at[pl.ds(sid * ZR, ZR)])
        plsc.subcore_barrier()

        if mode == 0:
            ci0 = (sid * 2 + cid) * nch
        else:
            ci0 = sid * nch
            mf = jnp.where(cid == 0, 1.0, 0.0)
            o0 = jnp.where(cid == 0, 0, 32)
            o1 = jnp.where(cid == 0, 16, 48)
            o2 = jnp.where(cid == 0, 32, 0)
            o3 = jnp.where(cid == 0, 48, 16)

        def ed_src(c):
            return ed_hbm.at[pl.ds((ci0 + c) * 5, 5)]

        pltpu.sync_copy(ed_src(0), ed0)
        pltpu.async_copy(y_hbm.at[ed0.at[0]], rw0, sg0)
        pltpu.async_copy(ed_src(1), ed1, se1)

        def compute_chunk(ed_v, rows_v, msg_v):
            def group_body(g, carry):
                c0g = lax.bitcast_convert_type(ed_v[2, pl.ds(g * 16, 16)], _F32)
                c1g = lax.bitcast_convert_type(ed_v[3, pl.ds(g * 16, 16)], _F32)
                c2g = lax.bitcast_convert_type(ed_v[4, pl.ds(g * 16, 16)], _F32)
                if mode != 0:
                    pa = c0g * mf + c1g * (1.0 - mf)
                    pb = c0g * mf + c2g * (1.0 - mf)
                    pc = c1g * mf + c2g * (1.0 - mf)
                for t in range(16):
                    e = g * 16 + t
                    if mode == 0:
                        c0 = c0g[t]
                        c1 = c1g[t]
                        c2 = c2g[t]
                        for j in range(4):
                            msg_v[e, pl.ds(j * 16, 16)] = (
                                c0 * rows_v[e, pl.ds(j * 16, 16)]
                                + c1 * rows_v[e, pl.ds(64 + j * 16, 16)]
                                + c2 * rows_v[e, pl.ds(128 + j * 16, 16)])
                    else:
                        for p, (cg, off) in enumerate(
                                ((pa, o0), (pa, o1), (pb, o2),
                                 (pb, o3), (pc, o0), (pc, o1))):
                            msg_v[e, pl.ds(p * 16, 16)] = (
                                cg[t] * rows_v[e, pl.ds(off, 16)])
                return carry

            lax.fori_loop(0, CH // 16, group_body, 0)

        def step(s, carry):
            for b in range(4):
                c = s * 4 + b
                gb = b % 2
                nb = (b + 1) % 2
                @pl.when(c + 1 < nch)
                def _():
                    pltpu.make_async_copy(
                        ed_src(c + 1), eds[(b + 1) % 4],
                        ses[(b + 1) % 4]).wait()
                    pltpu.async_copy(
                        y_hbm.at[eds[(b + 1) % 4].at[0]], rows[nb], sgs[nb])

                pltpu.make_async_copy(
                    y_hbm.at[eds[b].at[0]], rows[gb], sgs[gb]).wait()

                @pl.when(c >= 2)
                def _():
                    pltpu.make_async_copy(
                        msgs[gb], acc_sh.at[eds[(b + 2) % 4].at[1]],
                        sss[gb]).wait()

                @pl.when(c + 2 < nch)
                def _():
                    pltpu.async_copy(ed_src(c + 2), eds[(b + 2) % 4],
                                     ses[(b + 2) % 4])

                compute_chunk(eds[b], rows[gb], msgs[gb])
                pltpu.async_copy(msgs[gb], acc_sh.at[eds[b].at[1]],
                                 sss[gb], add=True)
            return carry

        lax.fori_loop(0, nch // 4, step, 0)
        pltpu.make_async_copy(msgs[0], acc_sh.at[eds[2].at[1]], sss[0]).wait()
        pltpu.make_async_copy(msgs[1], acc_sh.at[eds[3].at[1]], sss[1]).wait()
        plsc.subcore_barrier()
        pltpu.sync_copy(acc_sh.at[pl.ds(sid * ZR, ZR)],
                        acc_hbm.at[cid, pl.ds(sid * ZR, ZR)])

    return scgs


@functools.lru_cache(maxsize=None)
def _get_scgs(mode):
    return _make_scgs(mode)


def _scgs_wide(*args):
    return _get_scgs(0)(*args)


def _scgs_outer(*args):
    return _get_scgs(1)(*args)


def _pack_ed(src_r, dst_r, c0, c1, c2):
    cb = [lax.bitcast_convert_type(c.reshape(EPAD // CH, CH), jnp.int32)
          for c in (c0, c1, c2)]
    return jnp.stack([src_r, dst_r, *cb], axis=1).reshape(EPAD // CH * 5, CH)


def kernel(x, edge_index, edge_vec, Wl0, Wself0, fc1_0, fc2_0,
           Wl1, Wself1, fc1_1, fc2_1, Wl2, Wself2, fc1_2, fc2_2):
    pad_e = EPAD - N_EDGES
    src = jnp.concatenate([edge_index[0], jnp.zeros((pad_e,), jnp.int32)])
    dst = jnp.concatenate([edge_index[1], jnp.zeros((pad_e,), jnp.int32)])
    evt = jnp.pad(edge_vec.T, ((0, 0), (0, pad_e)))

    wcat0 = jnp.concatenate([Wl0[0], Wl0[1], Wl0[2], Wself0], axis=1)
    wcat1 = jnp.concatenate([Wl1[0], Wl1[1], Wl1[2], Wself1], axis=1)
    w2stack = jnp.concatenate([Wl2[0], Wl2[1], Wl2[2]], axis=0)
    zeros64 = jnp.zeros((NPAD, 64), _F32)
    zeros96 = jnp.zeros((NPAD, 96), _F32)

    c_all = _coef_call(evt,
                       (fc1_0.T, fc1_1.T, fc1_2.T),
                       (fc2_0.T, fc2_1.T, fc2_2.T))

    src_r = src.reshape(EPAD // CH, CH)
    dst_r = dst.reshape(EPAD // CH, CH)
    ed_l0 = _pack_ed(src_r, dst_r, c_all[0], c_all[1], c_all[2])
    ed_l1 = _pack_ed(src_r, dst_r, c_all[3], c_all[4], c_all[5])
    ed_l2 = _pack_ed(src_r, dst_r, c_all[6], c_all[7], c_all[8])

    y0, s0 = _mm0_call(x, wcat0)
    acc0 = _scgs_wide(y0, ed_l0, zeros64)
    y1, s1 = _mmc_call(s0, acc0, wcat1)
    acc1 = _scgs_wide(y1, ed_l1, zeros64)
    h2 = _comb2_call(s1, acc1)
    a2 = _scgs_outer(h2, ed_l2, zeros96)
    return _final_call(h2, a2, Wself2, w2stack)


# speedup vs baseline: 2.9130x; 1.0694x over previous
import functools

import jax
import jax.numpy as jnp
import numpy as np
from jax import lax
from jax.experimental import pallas as pl
from jax.experimental.pallas import tpu as pltpu
from jax.experimental.pallas import tpu_sc as plsc

N_NODES = 10000
N_EDGES = 160000
NPAD = 10240
EPAD = 163840
CH = 128
EPW = EPAD // 32
NCH = EPW // CH
ZR = NPAD // 16

_F32 = jnp.float32
_CPT_CORE0 = 52
_STEP = 3.5 / 11.0
_EMB_SCALE = 1.14136 * float(np.exp(2.0)) * float(np.sqrt(10.0))


def _coef_body(ev_ref, f1t0, f2t0, f1t1, f2t1, f1t2, f2t2, c_ref):
    ev = ev_ref[...]
    x = ev[0:1, :]
    y = ev[1:2, :]
    z = ev[2:3, :]
    r2 = x * x + y * y + z * z
    r = jnp.sqrt(r2 + 1e-12)
    inv = 1.0 / r
    ux, uy, uz = x * inv, y * inv, z * inv
    s1 = 1.7320508075688772 * (ux + uy + uz)
    s2 = (3.872983346207417 * (ux * uy + uy * uz + ux * uz)
          + 1.118033988749895 * (3.0 * uz * uz - 1.0)
          + 1.9364916731037085 * (ux * ux - uy * uy))
    s_all = jnp.concatenate([jnp.ones_like(s1), s1, s2], axis=0)

    vals = (lax.broadcasted_iota(jnp.int32, (10, 1), 0).astype(_F32) + 1.0) * _STEP
    diff = (r - vals) * (1.0 / _STEP)

    def _sus(t):
        return jnp.where(t > 0.0, jnp.exp(-1.0 / jnp.clip(t, 1e-8, None)), 0.0)

    emb = _EMB_SCALE * _sus(diff + 1.0) * _sus(1.0 - diff)

    rows = []
    for f1t, f2t in ((f1t0, f2t0), (f1t1, f2t1), (f1t2, f2t2)):
        hid = jnp.dot(f1t[...], emb, preferred_element_type=_F32)
        hid = hid * (1.0 / (1.0 + jnp.exp(-hid)))
        w = jnp.dot(f2t[...], hid, preferred_element_type=_F32)
        rows.append(w * s_all)
    c = jnp.concatenate(rows, axis=0)
    c_ref[...] = jnp.concatenate([c, jnp.zeros_like(c[0:7, :])], axis=0)


def _coef_call(evt, f1ts, f2ts):
    be = 8192
    grid = (EPAD // be,)
    full = pl.BlockSpec((None, None), lambda i: (0, 0))
    specs = [pl.BlockSpec((3, be), lambda i: (0, i))]
    for _ in range(3):
        specs.append(pl.BlockSpec((100, 10), lambda i: (0, 0)))
        specs.append(pl.BlockSpec((3, 100), lambda i: (0, 0)))
    args = [evt]
    for f1t, f2t in zip(f1ts, f2ts):
        args += [f1t, f2t]
    return pl.pallas_call(
        _coef_body,
        grid=grid,
        in_specs=specs,
        out_specs=pl.BlockSpec((16, be), lambda i: (0, i)),
        out_shape=jax.ShapeDtypeStruct((16, EPAD), _F32),
    )(*args)


def _mm0_body(x_ref, w_ref, y_ref, s_ref):
    t = jnp.dot(x_ref[...], w_ref[...], preferred_element_type=_F32)
    y_ref[...] = t[:, :192]
    s_ref[...] = t[:, 192:]


def _mm0_call(x, wcat):
    br = 2000
    return pl.pallas_call(
        _mm0_body,
        grid=(N_NODES // br,),
        in_specs=[pl.BlockSpec((br, 256), lambda i: (i, 0)),
                  pl.BlockSpec((256, 256), lambda i: (0, 0))],
        out_specs=[pl.BlockSpec((br, 192), lambda i: (i, 0)),
                   pl.BlockSpec((br, 64), lambda i: (i, 0))],
        out_shape=[jax.ShapeDtypeStruct((N_NODES, 192), _F32),
                   jax.ShapeDtypeStruct((N_NODES, 64), _F32)],
    )(x, wcat)


def _mmc_body(s_ref, acc_ref, w_ref, y_ref, sn_ref):
    pre = s_ref[...] + 0.25 * (acc_ref[0] + acc_ref[1])
    h = pre * (1.0 / (1.0 + jnp.exp(-pre)))
    t = jnp.dot(h, w_ref[...], preferred_element_type=_F32)
    y_ref[...] = t[:, :192]
    sn_ref[...] = t[:, 192:]


def _mmc_call(s_prev, acc, wcat):
    br = 2000
    return pl.pallas_call(
        _mmc_body,
        grid=(N_NODES // br,),
        in_specs=[pl.BlockSpec((br, 64), lambda i: (i, 0)),
                  pl.BlockSpec((2, br, 64), lambda i: (0, i, 0)),
                  pl.BlockSpec((64, 256), lambda i: (0, 0))],
        out_specs=[pl.BlockSpec((br, 192), lambda i: (i, 0)),
                   pl.BlockSpec((br, 64), lambda i: (i, 0))],
        out_shape=[jax.ShapeDtypeStruct((N_NODES, 192), _F32),
                   jax.ShapeDtypeStruct((N_NODES, 64), _F32)],
    )(s_prev, acc, wcat)


def _comb2_body(s_ref, acc_ref, h_ref):
    pre = s_ref[...] + 0.25 * (acc_ref[0] + acc_ref[1])
    h_ref[...] = pre * (1.0 / (1.0 + jnp.exp(-pre)))


def _comb2_call(s_prev, acc):
    br = 2000
    return pl.pallas_call(
        _comb2_body,
        grid=(N_NODES // br,),
        in_specs=[pl.BlockSpec((br, 64), lambda i: (i, 0)),
                  pl.BlockSpec((2, br, 64), lambda i: (0, i, 0))],
        out_specs=pl.BlockSpec((br, 64), lambda i: (i, 0)),
        out_shape=jax.ShapeDtypeStruct((N_NODES, 64), _F32),
    )(s_prev, acc)


def _final_body(h_ref, a_ref, ws_ref, w2_ref, o_ref):
    a = jnp.concatenate([a_ref[0], a_ref[1]], axis=1)
    o_ref[...] = (jnp.dot(h_ref[...], ws_ref[...], preferred_element_type=_F32)
                  + 0.25 * jnp.dot(a, w2_ref[...], preferred_element_type=_F32))


def _final_call(h2, acc, wself2, w2stack):
    br = 2000
    return pl.pallas_call(
        _final_body,
        grid=(N_NODES // br,),
        in_specs=[pl.BlockSpec((br, 64), lambda i: (i, 0)),
                  pl.BlockSpec((2, br, 96), lambda i: (0, i, 0)),
                  pl.BlockSpec((64, 256), lambda i: (0, 0)),
                  pl.BlockSpec((192, 256), lambda i: (0, 0))],
        out_specs=pl.BlockSpec((br, 256), lambda i: (i, 0)),
        out_shape=jax.ShapeDtypeStruct((N_NODES, 256), _F32),
    )(h2, acc, wself2, w2stack)


def _make_scgs(mode):
    gw = 192 if mode == 0 else 64
    sw = 64 if mode == 0 else 96
    nch = NCH if mode == 0 else EPAD // 16 // CH
    mesh = plsc.VectorSubcoreMesh(core_axis_name="c", subcore_axis_name="s",
                                  num_cores=2, num_subcores=16)

    @functools.partial(
        pl.kernel,
        out_type=jax.ShapeDtypeStruct((2, NPAD, sw), _F32),
        mesh=mesh,
        compiler_params=pltpu.CompilerParams(use_tc_tiling_on_sc=False),
        scratch_types=[
            pltpu.VMEM((5, CH), jnp.int32),
            pltpu.VMEM((5, CH), jnp.int32),
            pltpu.VMEM((5, CH), jnp.int32),
            pltpu.VMEM((5, CH), jnp.int32),
            pltpu.VMEM((CH, gw), _F32),
            pltpu.VMEM((CH, gw), _F32),
            pltpu.VMEM((CH, sw), _F32),
            pltpu.VMEM((CH, sw), _F32),
            pltpu.VMEM_SHARED((NPAD, sw), _F32),
            pltpu.SemaphoreType.DMA,
            pltpu.SemaphoreType.DMA,
            pltpu.SemaphoreType.DMA,
            pltpu.SemaphoreType.DMA,
            pltpu.SemaphoreType.DMA,
            pltpu.SemaphoreType.DMA,
            pltpu.SemaphoreType.DMA,
            pltpu.SemaphoreType.DMA,
        ],
    )
    def scgs(y_hbm, ed_hbm, z_hbm, acc_hbm,
             ed0, ed1, ed2, ed3, rw0, rw1, mg0, mg1, acc_sh,
             sg0, sg1, se0, se1, se2, se3, ss0, ss1):
        cid = lax.axis_index("c")
        sid = lax.axis_index("s")
        eds = (ed0, ed1, ed2, ed3)
        rows = (rw0, rw1)
        msgs = (mg0, mg1)
        sgs = (sg0, sg1)
        ses = (se0, se1, se2, se3)
        sss = (ss0, ss1)

        pltpu.sync_copy(z_hbm.at[pl.ds(sid * ZR, ZR)],
                        acc_sh.at[pl.ds(sid * ZR, ZR)])
        plsc.subcore_barrier()

        if mode == 0:
            cpt0 = _CPT_CORE0
            cpt1 = (EPAD // CH - 16 * cpt0) // 16
            nch_w = jnp.where(cid == 0, cpt0, cpt1)
            ci0 = jnp.where(cid == 0, sid * cpt0, 16 * cpt0 + sid * cpt1)
        else:
            nch_w = nch
            ci0 = sid * nch
            mf = jnp.where(cid == 0, 1.0, 0.0)
            o0 = jnp.where(cid == 0, 0, 32)
            o1 = jnp.where(cid == 0, 16, 48)
            o2 = jnp.where(cid == 0, 32, 0)
            o3 = jnp.where(cid == 0, 48, 16)

        def ed_src(c):
            return ed_hbm.at[pl.ds((ci0 + c) * 5, 5)]

        pltpu.sync_copy(ed_src(0), ed0)
        pltpu.async_copy(y_hbm.at[ed0.at[0]], rw0, sg0)
        pltpu.async_copy(ed_src(1), ed1, se1)

        def compute_chunk(ed_v, rows_v, msg_v):
            def group_body(g, carry):
                c0g = lax.bitcast_convert_type(ed_v[2, pl.ds(g * 16, 16)], _F32)
                c1g = lax.bitcast_convert_type(ed_v[3, pl.ds(g * 16, 16)], _F32)
                c2g = lax.bitcast_convert_type(ed_v[4, pl.ds(g * 16, 16)], _F32)
                if mode != 0:
                    pa = c0g * mf + c1g * (1.0 - mf)
                    pb = c0g * mf + c2g * (1.0 - mf)
                    pc = c1g * mf + c2g * (1.0 - mf)
                for t in range(16):
                    e = g * 16 + t
                    if mode == 0:
                        c0 = c0g[t]
                        c1 = c1g[t]
                        c2 = c2g[t]
                        for j in range(4):
                            msg_v[e, pl.ds(j * 16, 16)] = (
                                c0 * rows_v[e, pl.ds(j * 16, 16)]
                                + c1 * rows_v[e, pl.ds(64 + j * 16, 16)]
                                + c2 * rows_v[e, pl.ds(128 + j * 16, 16)])
                    else:
                        for p, (cg, off) in enumerate(
                                ((pa, o0), (pa, o1), (pb, o2),
                                 (pb, o3), (pc, o0), (pc, o1))):
                            msg_v[e, pl.ds(p * 16, 16)] = (
                                cg[t] * rows_v[e, pl.ds(off, 16)])
                return carry

            lax.fori_loop(0, CH // 16, group_body, 0)

        def step(s, carry):
            for b in range(4):
                c = s * 4 + b
                gb = b % 2
                nb = (b + 1) % 2
                @pl.when(c + 1 < nch_w)
                def _():
                    pltpu.make_async_copy(
                        ed_src(c + 1), eds[(b + 1) % 4],
                        ses[(b + 1) % 4]).wait()
                    pltpu.async_copy(
                        y_hbm.at[eds[(b + 1) % 4].at[0]], rows[nb], sgs[nb])

                pltpu.make_async_copy(
                    y_hbm.at[eds[b].at[0]], rows[gb], sgs[gb]).wait()

                @pl.when(c >= 2)
                def _():
                    pltpu.make_async_copy(
                        msgs[gb], acc_sh.at[eds[(b + 2) % 4].at[1]],
                        sss[gb]).wait()

                @pl.when(c + 2 < nch_w)
                def _():
                    pltpu.async_copy(ed_src(c + 2), eds[(b + 2) % 4],
                                     ses[(b + 2) % 4])

                compute_chunk(eds[b], rows[gb], msgs[gb])
                pltpu.async_copy(msgs[gb], acc_sh.at[eds[b].at[1]],
                                 sss[gb], add=True)
            return carry

        lax.fori_loop(0, nch_w // 4, step, 0)
        pltpu.make_async_copy(msgs[0], acc_sh.at[eds[2].at[1]], sss[0]).wait()
        pltpu.make_async_copy(msgs[1], acc_sh.at[eds[3].at[1]], sss[1]).wait()
        plsc.subcore_barrier()
        pltpu.sync_copy(acc_sh.at[pl.ds(sid * ZR, ZR)],
                        acc_hbm.at[cid, pl.ds(sid * ZR, ZR)])

    return scgs


@functools.lru_cache(maxsize=None)
def _get_scgs(mode):
    return _make_scgs(mode)


def _scgs_wide(*args):
    return _get_scgs(0)(*args)


def _scgs_outer(*args):
    return _get_scgs(1)(*args)


def _pack_ed(src_r, dst_r, c0, c1, c2):
    cb = [lax.bitcast_convert_type(c.reshape(EPAD // CH, CH), jnp.int32)
          for c in (c0, c1, c2)]
    return jnp.stack([src_r, dst_r, *cb], axis=1).reshape(EPAD // CH * 5, CH)


def kernel(x, edge_index, edge_vec, Wl0, Wself0, fc1_0, fc2_0,
           Wl1, Wself1, fc1_1, fc2_1, Wl2, Wself2, fc1_2, fc2_2):
    pad_e = EPAD - N_EDGES
    src = jnp.concatenate([edge_index[0], jnp.zeros((pad_e,), jnp.int32)])
    dst = jnp.concatenate([edge_index[1], jnp.zeros((pad_e,), jnp.int32)])
    evt = jnp.pad(edge_vec.T, ((0, 0), (0, pad_e)))

    wcat0 = jnp.concatenate([Wl0[0], Wl0[1], Wl0[2], Wself0], axis=1)
    wcat1 = jnp.concatenate([Wl1[0], Wl1[1], Wl1[2], Wself1], axis=1)
    w2stack = jnp.concatenate([Wl2[0], Wl2[1], Wl2[2]], axis=0)
    zeros64 = jnp.zeros((NPAD, 64), _F32)
    zeros96 = jnp.zeros((NPAD, 96), _F32)

    c_all = _coef_call(evt,
                       (fc1_0.T, fc1_1.T, fc1_2.T),
                       (fc2_0.T, fc2_1.T, fc2_2.T))

    src_r = src.reshape(EPAD // CH, CH)
    dst_r = dst.reshape(EPAD // CH, CH)
    ed_l0 = _pack_ed(src_r, dst_r, c_all[0], c_all[1], c_all[2])
    ed_l1 = _pack_ed(src_r, dst_r, c_all[3], c_all[4], c_all[5])
    ed_l2 = _pack_ed(src_r, dst_r, c_all[6], c_all[7], c_all[8])

    y0, s0 = _mm0_call(x, wcat0)
    acc0 = _scgs_wide(y0, ed_l0, zeros64)
    y1, s1 = _mmc_call(s0, acc0, wcat1)
    acc1 = _scgs_wide(y1, ed_l1, zeros64)
    h2 = _comb2_call(s1, acc1)
    a2 = _scgs_outer(h2, ed_l2, zeros96)
    return _final_call(h2, a2, Wself2, w2stack)


# speedup vs baseline: 2.9610x; 1.0165x over previous
import functools

import jax
import jax.numpy as jnp
import numpy as np
from jax import lax
from jax.experimental import pallas as pl
from jax.experimental.pallas import tpu as pltpu
from jax.experimental.pallas import tpu_sc as plsc

N_NODES = 10000
N_EDGES = 160000
NPAD = 10240
EPAD = 163840
CH = 128
EPW = EPAD // 32
NCH = EPW // CH
ZR = NPAD // 16

_F32 = jnp.float32
_CPT_CORE0 = 52
_STEP = 3.5 / 11.0
_EMB_SCALE = 1.14136 * float(np.exp(2.0)) * float(np.sqrt(10.0))


def _coef_body(ev_ref, f1t0, f2t0, f1t1, f2t1, f1t2, f2t2, c_ref):
    ev = ev_ref[...]
    x = ev[0:1, :]
    y = ev[1:2, :]
    z = ev[2:3, :]
    r2 = x * x + y * y + z * z
    r = jnp.sqrt(r2 + 1e-12)
    inv = 1.0 / r
    ux, uy, uz = x * inv, y * inv, z * inv
    s1 = 1.7320508075688772 * (ux + uy + uz)
    s2 = (3.872983346207417 * (ux * uy + uy * uz + ux * uz)
          + 1.118033988749895 * (3.0 * uz * uz - 1.0)
          + 1.9364916731037085 * (ux * ux - uy * uy))
    s_all = jnp.concatenate([jnp.ones_like(s1), s1, s2], axis=0)

    vals = (lax.broadcasted_iota(jnp.int32, (10, 1), 0).astype(_F32) + 1.0) * _STEP
    diff = (r - vals) * (1.0 / _STEP)

    def _sus(t):
        return jnp.where(t > 0.0, jnp.exp(-1.0 / jnp.clip(t, 1e-8, None)), 0.0)

    emb = _EMB_SCALE * _sus(diff + 1.0) * _sus(1.0 - diff)

    rows = []
    for f1t, f2t in ((f1t0, f2t0), (f1t1, f2t1), (f1t2, f2t2)):
        hid = jnp.dot(f1t[...], emb, preferred_element_type=_F32)
        hid = hid * (0.5 + 0.5 * jnp.tanh(0.5 * hid))
        w = jnp.dot(f2t[...], hid, preferred_element_type=_F32)
        rows.append(w * s_all)
    c = jnp.concatenate(rows, axis=0)
    c_ref[...] = jnp.concatenate([c, jnp.zeros_like(c[0:7, :])], axis=0)


def _coef_call(evt, f1ts, f2ts):
    be = 8192
    grid = (EPAD // be,)
    full = pl.BlockSpec((None, None), lambda i: (0, 0))
    specs = [pl.BlockSpec((3, be), lambda i: (0, i))]
    for _ in range(3):
        specs.append(pl.BlockSpec((100, 10), lambda i: (0, 0)))
        specs.append(pl.BlockSpec((3, 100), lambda i: (0, 0)))
    args = [evt]
    for f1t, f2t in zip(f1ts, f2ts):
        args += [f1t, f2t]
    return pl.pallas_call(
        _coef_body,
        grid=grid,
        in_specs=specs,
        out_specs=pl.BlockSpec((16, be), lambda i: (0, i)),
        out_shape=jax.ShapeDtypeStruct((16, EPAD), _F32),
    )(*args)


def _mm0_body(x_ref, w_ref, y_ref, s_ref):
    t = jnp.dot(x_ref[...], w_ref[...], preferred_element_type=_F32)
    y_ref[...] = t[:, :192]
    s_ref[...] = t[:, 192:]


def _mm0_call(x, wcat):
    br = 2000
    return pl.pallas_call(
        _mm0_body,
        grid=(N_NODES // br,),
        in_specs=[pl.BlockSpec((br, 256), lambda i: (i, 0)),
                  pl.BlockSpec((256, 256), lambda i: (0, 0))],
        out_specs=[pl.BlockSpec((br, 192), lambda i: (i, 0)),
                   pl.BlockSpec((br, 64), lambda i: (i, 0))],
        out_shape=[jax.ShapeDtypeStruct((N_NODES, 192), _F32),
                   jax.ShapeDtypeStruct((N_NODES, 64), _F32)],
    )(x, wcat)


def _mmc_body(s_ref, acc_ref, w_ref, y_ref, sn_ref):
    pre = s_ref[...] + 0.25 * (acc_ref[0] + acc_ref[1])
    h = pre * (0.5 + 0.5 * jnp.tanh(0.5 * pre))
    t = jnp.dot(h, w_ref[...], preferred_element_type=_F32)
    y_ref[...] = t[:, :192]
    sn_ref[...] = t[:, 192:]


def _mmc_call(s_prev, acc, wcat):
    br = 2000
    return pl.pallas_call(
        _mmc_body,
        grid=(N_NODES // br,),
        in_specs=[pl.BlockSpec((br, 64), lambda i: (i, 0)),
                  pl.BlockSpec((2, br, 64), lambda i: (0, i, 0)),
                  pl.BlockSpec((64, 256), lambda i: (0, 0))],
        out_specs=[pl.BlockSpec((br, 192), lambda i: (i, 0)),
                   pl.BlockSpec((br, 64), lambda i: (i, 0))],
        out_shape=[jax.ShapeDtypeStruct((N_NODES, 192), _F32),
                   jax.ShapeDtypeStruct((N_NODES, 64), _F32)],
    )(s_prev, acc, wcat)


def _comb2_body(s_ref, acc_ref, h_ref):
    pre = s_ref[...] + 0.25 * (acc_ref[0] + acc_ref[1])
    h_ref[...] = pre * (0.5 + 0.5 * jnp.tanh(0.5 * pre))


def _comb2_call(s_prev, acc):
    br = 2000
    return pl.pallas_call(
        _comb2_body,
        grid=(N_NODES // br,),
        in_specs=[pl.BlockSpec((br, 64), lambda i: (i, 0)),
                  pl.BlockSpec((2, br, 64), lambda i: (0, i, 0))],
        out_specs=pl.BlockSpec((br, 64), lambda i: (i, 0)),
        out_shape=jax.ShapeDtypeStruct((N_NODES, 64), _F32),
    )(s_prev, acc)


def _final_body(h_ref, a_ref, ws_ref, w2_ref, o_ref):
    a = jnp.concatenate([a_ref[0], a_ref[1]], axis=1)
    o_ref[...] = (jnp.dot(h_ref[...], ws_ref[...], preferred_element_type=_F32)
                  + 0.25 * jnp.dot(a, w2_ref[...], preferred_element_type=_F32))


def _final_call(h2, acc, wself2, w2stack):
    br = 2000
    return pl.pallas_call(
        _final_body,
        grid=(N_NODES // br,),
        in_specs=[pl.BlockSpec((br, 64), lambda i: (i, 0)),
                  pl.BlockSpec((2, br, 96), lambda i: (0, i, 0)),
                  pl.BlockSpec((64, 256), lambda i: (0, 0)),
                  pl.BlockSpec((192, 256), lambda i: (0, 0))],
        out_specs=pl.BlockSpec((br, 256), lambda i: (i, 0)),
        out_shape=jax.ShapeDtypeStruct((N_NODES, 256), _F32),
    )(h2, acc, wself2, w2stack)


def _make_scgs(mode):
    gw = 192 if mode == 0 else 64
    sw = 64 if mode == 0 else 96
    nch = NCH if mode == 0 else EPAD // 16 // CH
    mesh = plsc.VectorSubcoreMesh(core_axis_name="c", subcore_axis_name="s",
                                  num_cores=2, num_subcores=16)

    @functools.partial(
        pl.kernel,
        out_type=jax.ShapeDtypeStruct((2, NPAD, sw), _F32),
        mesh=mesh,
        compiler_params=pltpu.CompilerParams(use_tc_tiling_on_sc=False),
        scratch_types=[
            pltpu.VMEM((5, CH), jnp.int32),
            pltpu.VMEM((5, CH), jnp.int32),
            pltpu.VMEM((5, CH), jnp.int32),
            pltpu.VMEM((5, CH), jnp.int32),
            pltpu.VMEM((CH, gw), _F32),
            pltpu.VMEM((CH, gw), _F32),
            pltpu.VMEM((CH, sw), _F32),
            pltpu.VMEM((CH, sw), _F32),
            pltpu.VMEM_SHARED((NPAD, sw), _F32),
            pltpu.SemaphoreType.DMA,
            pltpu.SemaphoreType.DMA,
            pltpu.SemaphoreType.DMA,
            pltpu.SemaphoreType.DMA,
            pltpu.SemaphoreType.DMA,
            pltpu.SemaphoreType.DMA,
            pltpu.SemaphoreType.DMA,
            pltpu.SemaphoreType.DMA,
        ],
    )
    def scgs(y_hbm, ed_hbm, z_hbm, acc_hbm,
             ed0, ed1, ed2, ed3, rw0, rw1, mg0, mg1, acc_sh,
             sg0, sg1, se0, se1, se2, se3, ss0, ss1):
        cid = lax.axis_index("c")
        sid = lax.axis_index("s")
        eds = (ed0, ed1, ed2, ed3)
        rows = (rw0, rw1)
        msgs = (mg0, mg1)
        sgs = (sg0, sg1)
        ses = (se0, se1, se2, se3)
        sss = (ss0, ss1)

        pltpu.sync_copy(z_hbm.at[pl.ds(sid * ZR, ZR)],
                        acc_sh.at[pl.ds(sid * ZR, ZR)])
        plsc.subcore_barrier()

        if mode == 0:
            cpt0 = _CPT_CORE0
            cpt1 = (EPAD // CH - 16 * cpt0) // 16
            nch_w = jnp.where(cid == 0, cpt0, cpt1)
            ci0 = jnp.where(cid == 0, sid * cpt0, 16 * cpt0 + sid * cpt1)
        else:
            nch_w = nch
            ci0 = sid * nch
            mf = jnp.where(cid == 0, 1.0, 0.0)
            o0 = jnp.where(cid == 0, 0, 32)
            o1 = jnp.where(cid == 0, 16, 48)
            o2 = jnp.where(cid == 0, 32, 0)
            o3 = jnp.where(cid == 0, 48, 16)

        def ed_src(c):
            return ed_hbm.at[pl.ds((ci0 + c) * 5, 5)]

        pltpu.sync_copy(ed_src(0), ed0)
        pltpu.async_copy(y_hbm.at[ed0.at[0]], rw0, sg0)
        pltpu.async_copy(ed_src(1), ed1, se1)

        def compute_chunk(ed_v, rows_v, msg_v):
            def group_body(g, carry):
                c0g = lax.bitcast_convert_type(ed_v[2, pl.ds(g * 16, 16)], _F32)
                c1g = lax.bitcast_convert_type(ed_v[3, pl.ds(g * 16, 16)], _F32)
                c2g = lax.bitcast_convert_type(ed_v[4, pl.ds(g * 16, 16)], _F32)
                if mode != 0:
                    pa = c0g * mf + c1g * (1.0 - mf)
                    pb = c0g * mf + c2g * (1.0 - mf)
                    pc = c1g * mf + c2g * (1.0 - mf)
                for t in range(16):
                    e = g * 16 + t
                    if mode == 0:
                        c0 = c0g[t]
                        c1 = c1g[t]
                        c2 = c2g[t]
                        for j in range(4):
                            msg_v[e, pl.ds(j * 16, 16)] = (
                                c0 * rows_v[e, pl.ds(j * 16, 16)]
                                + c1 * rows_v[e, pl.ds(64 + j * 16, 16)]
                                + c2 * rows_v[e, pl.ds(128 + j * 16, 16)])
                    else:
                        for p, (cg, off) in enumerate(
                                ((pa, o0), (pa, o1), (pb, o2),
                                 (pb, o3), (pc, o0), (pc, o1))):
                            msg_v[e, pl.ds(p * 16, 16)] = (
                                cg[t] * rows_v[e, pl.ds(off, 16)])
                return carry

            lax.fori_loop(0, CH // 16, group_body, 0)

        def step(s, carry):
            for b in range(4):
                c = s * 4 + b
                gb = b % 2
                nb = (b + 1) % 2
                @pl.when(c + 1 < nch_w)
                def _():
                    pltpu.make_async_copy(
                        ed_src(c + 1), eds[(b + 1) % 4],
                        ses[(b + 1) % 4]).wait()
                    pltpu.async_copy(
                        y_hbm.at[eds[(b + 1) % 4].at[0]], rows[nb], sgs[nb])

                pltpu.make_async_copy(
                    y_hbm.at[eds[b].at[0]], rows[gb], sgs[gb]).wait()

                @pl.when(c >= 2)
                def _():
                    pltpu.make_async_copy(
                        msgs[gb], acc_sh.at[eds[(b + 2) % 4].at[1]],
                        sss[gb]).wait()

                @pl.when(c + 2 < nch_w)
                def _():
                    pltpu.async_copy(ed_src(c + 2), eds[(b + 2) % 4],
                                     ses[(b + 2) % 4])

                compute_chunk(eds[b], rows[gb], msgs[gb])
                pltpu.async_copy(msgs[gb], acc_sh.at[eds[b].at[1]],
                                 sss[gb], add=True)
            return carry

        lax.fori_loop(0, nch_w // 4, step, 0)
        pltpu.make_async_copy(msgs[0], acc_sh.at[eds[2].at[1]], sss[0]).wait()
        pltpu.make_async_copy(msgs[1], acc_sh.at[eds[3].at[1]], sss[1]).wait()
        plsc.subcore_barrier()
        pltpu.sync_copy(acc_sh.at[pl.ds(sid * ZR, ZR)],
                        acc_hbm.at[cid, pl.ds(sid * ZR, ZR)])

    return scgs


@functools.lru_cache(maxsize=None)
def _get_scgs(mode):
    return _make_scgs(mode)


def _scgs_wide(*args):
    return _get_scgs(0)(*args)


def _scgs_outer(*args):
    return _get_scgs(1)(*args)


def _pack_ed(src_r, dst_r, c0, c1, c2):
    cb = [lax.bitcast_convert_type(c.reshape(EPAD // CH, CH), jnp.int32)
          for c in (c0, c1, c2)]
    return jnp.stack([src_r, dst_r, *cb], axis=1).reshape(EPAD // CH * 5, CH)


def kernel(x, edge_index, edge_vec, Wl0, Wself0, fc1_0, fc2_0,
           Wl1, Wself1, fc1_1, fc2_1, Wl2, Wself2, fc1_2, fc2_2):
    pad_e = EPAD - N_EDGES
    src = jnp.concatenate([edge_index[0], jnp.zeros((pad_e,), jnp.int32)])
    dst = jnp.concatenate([edge_index[1], jnp.zeros((pad_e,), jnp.int32)])
    evt = jnp.pad(edge_vec.T, ((0, 0), (0, pad_e)))

    wcat0 = jnp.concatenate([Wl0[0], Wl0[1], Wl0[2], Wself0], axis=1)
    wcat1 = jnp.concatenate([Wl1[0], Wl1[1], Wl1[2], Wself1], axis=1)
    w2stack = jnp.concatenate([Wl2[0], Wl2[1], Wl2[2]], axis=0)
    zeros64 = jnp.zeros((NPAD, 64), _F32)
    zeros96 = jnp.zeros((NPAD, 96), _F32)

    c_all = _coef_call(evt,
                       (fc1_0.T, fc1_1.T, fc1_2.T),
                       (fc2_0.T, fc2_1.T, fc2_2.T))

    src_r = src.reshape(EPAD // CH, CH)
    dst_r = dst.reshape(EPAD // CH, CH)
    ed_l0 = _pack_ed(src_r, dst_r, c_all[0], c_all[1], c_all[2])
    ed_l1 = _pack_ed(src_r, dst_r, c_all[3], c_all[4], c_all[5])
    ed_l2 = _pack_ed(src_r, dst_r, c_all[6], c_all[7], c_all[8])

    y0, s0 = _mm0_call(x, wcat0)
    acc0 = _scgs_wide(y0, ed_l0, zeros64)
    y1, s1 = _mmc_call(s0, acc0, wcat1)
    acc1 = _scgs_wide(y1, ed_l1, zeros64)
    h2 = _comb2_call(s1, acc1)
    a2 = _scgs_outer(h2, ed_l2, zeros96)
    return _final_call(h2, a2, Wself2, w2stack)


# speedup vs baseline: 2.9886x; 1.0093x over previous
import functools

import jax
import jax.numpy as jnp
import numpy as np
from jax import lax
from jax.experimental import pallas as pl
from jax.experimental.pallas import tpu as pltpu
from jax.experimental.pallas import tpu_sc as plsc

N_NODES = 10000
N_EDGES = 160000
NPAD = 10240
EPAD = 163840
CH = 128
EPW = EPAD // 32
NCH = EPW // CH
ZR = NPAD // 16

_F32 = jnp.float32
_CPT_CORE0 = 52

_STEP = 3.5 / 11.0
_EMB_SCALE = 1.14136 * float(np.exp(2.0)) * float(np.sqrt(10.0))


def _coef_body(ev_ref, f1t0, f2t0, f1t1, f2t1, f1t2, f2t2, c_ref):
    ev = ev_ref[...]
    x = ev[0:1, :]
    y = ev[1:2, :]
    z = ev[2:3, :]
    r2 = x * x + y * y + z * z
    r = jnp.sqrt(r2 + 1e-12)
    inv = 1.0 / r
    ux, uy, uz = x * inv, y * inv, z * inv
    s1 = 1.7320508075688772 * (ux + uy + uz)
    s2 = (3.872983346207417 * (ux * uy + uy * uz + ux * uz)
          + 1.118033988749895 * (3.0 * uz * uz - 1.0)
          + 1.9364916731037085 * (ux * ux - uy * uy))
    s_all = jnp.concatenate([jnp.ones_like(s1), s1, s2], axis=0)

    vals = (lax.broadcasted_iota(jnp.int32, (10, 1), 0).astype(_F32) + 1.0) * _STEP
    diff = (r - vals) * (1.0 / _STEP)

    def _sus(t):
        return jnp.where(t > 0.0, jnp.exp(-1.0 / jnp.clip(t, 1e-8, None)), 0.0)

    emb = _EMB_SCALE * _sus(diff + 1.0) * _sus(1.0 - diff)

    rows = []
    for f1t, f2t in ((f1t0, f2t0), (f1t1, f2t1), (f1t2, f2t2)):
        hid = jnp.dot(f1t[...], emb, preferred_element_type=_F32)
        hid = hid * (0.5 + 0.5 * jnp.tanh(0.5 * hid))
        w = jnp.dot(f2t[...], hid, preferred_element_type=_F32)
        rows.append(w * s_all)
    c = jnp.concatenate(rows, axis=0)
    c_ref[...] = jnp.concatenate([c, jnp.zeros_like(c[0:7, :])], axis=0)


def _coef_call(evt, f1ts, f2ts):
    be = 8192
    grid = (EPAD // be,)
    full = pl.BlockSpec((None, None), lambda i: (0, 0))
    specs = [pl.BlockSpec((3, be), lambda i: (0, i))]
    for _ in range(3):
        specs.append(pl.BlockSpec((100, 10), lambda i: (0, 0)))
        specs.append(pl.BlockSpec((3, 100), lambda i: (0, 0)))
    args = [evt]
    for f1t, f2t in zip(f1ts, f2ts):
        args += [f1t, f2t]
    return pl.pallas_call(
        _coef_body,
        grid=grid,
        in_specs=specs,
        out_specs=pl.BlockSpec((16, be), lambda i: (0, i)),
        out_shape=jax.ShapeDtypeStruct((16, EPAD), _F32),
    )(*args)


def _mm0_body(x_ref, w_ref, y_ref, s_ref):
    t = jnp.dot(x_ref[...], w_ref[...], preferred_element_type=_F32)
    y_ref[...] = t[:, :192]
    s_ref[...] = t[:, 192:]


def _mm0_call(x, wcat):
    br = 2000
    return pl.pallas_call(
        _mm0_body,
        grid=(N_NODES // br,),
        in_specs=[pl.BlockSpec((br, 256), lambda i: (i, 0)),
                  pl.BlockSpec((256, 256), lambda i: (0, 0))],
        out_specs=[pl.BlockSpec((br, 192), lambda i: (i, 0)),
                   pl.BlockSpec((br, 64), lambda i: (i, 0))],
        out_shape=[jax.ShapeDtypeStruct((N_NODES, 192), _F32),
                   jax.ShapeDtypeStruct((N_NODES, 64), _F32)],
    )(x, wcat)


def _mmc_body(s_ref, acc_ref, w_ref, y_ref, sn_ref):
    pre = s_ref[...] + 0.25 * (acc_ref[0] + acc_ref[1])
    h = pre * (0.5 + 0.5 * jnp.tanh(0.5 * pre))
    t = jnp.dot(h, w_ref[...], preferred_element_type=_F32)
    y_ref[...] = t[:, :192]
    sn_ref[...] = t[:, 192:]


def _mmc_call(s_prev, acc, wcat):
    br = 2000
    return pl.pallas_call(
        _mmc_body,
        grid=(N_NODES // br,),
        in_specs=[pl.BlockSpec((br, 64), lambda i: (i, 0)),
                  pl.BlockSpec((2, br, 64), lambda i: (0, i, 0)),
                  pl.BlockSpec((64, 256), lambda i: (0, 0))],
        out_specs=[pl.BlockSpec((br, 192), lambda i: (i, 0)),
                   pl.BlockSpec((br, 64), lambda i: (i, 0))],
        out_shape=[jax.ShapeDtypeStruct((N_NODES, 192), _F32),
                   jax.ShapeDtypeStruct((N_NODES, 64), _F32)],
    )(s_prev, acc, wcat)


def _comb2_body(s_ref, acc_ref, h_ref, hb_ref):
    pre = s_ref[...] + 0.25 * (acc_ref[0] + acc_ref[1])
    h = pre * (0.5 + 0.5 * jnp.tanh(0.5 * pre))
    h_ref[...] = h
    hb_ref[...] = h.astype(jnp.bfloat16)


def _comb2_call(s_prev, acc):
    br = 2000
    return pl.pallas_call(
        _comb2_body,
        grid=(N_NODES // br,),
        in_specs=[pl.BlockSpec((br, 64), lambda i: (i, 0)),
                  pl.BlockSpec((2, br, 64), lambda i: (0, i, 0))],
        out_specs=[pl.BlockSpec((br, 64), lambda i: (i, 0)),
                   pl.BlockSpec((br, 64), lambda i: (i, 0))],
        out_shape=[jax.ShapeDtypeStruct((N_NODES, 64), _F32),
                   jax.ShapeDtypeStruct((N_NODES, 64), jnp.bfloat16)],
    )(s_prev, acc)


def _final_body(h_ref, a_ref, ws_ref, w2_ref, o_ref):
    a = a_ref[0].astype(_F32) + a_ref[1].astype(_F32)
    o_ref[...] = (jnp.dot(h_ref[...], ws_ref[...], preferred_element_type=_F32)
                  + 0.25 * jnp.dot(a, w2_ref[...], preferred_element_type=_F32))


def _final_call(h2, acc, wself2, w2stack):
    br = 2000
    return pl.pallas_call(
        _final_body,
        grid=(N_NODES // br,),
        in_specs=[pl.BlockSpec((br, 64), lambda i: (i, 0)),
                  pl.BlockSpec((2, br, 192), lambda i: (0, i, 0)),
                  pl.BlockSpec((64, 256), lambda i: (0, 0)),
                  pl.BlockSpec((192, 256), lambda i: (0, 0))],
        out_specs=pl.BlockSpec((br, 256), lambda i: (i, 0)),
        out_shape=jax.ShapeDtypeStruct((N_NODES, 256), _F32),
    )(h2, acc, wself2, w2stack)


def _make_scgs(mode):
    gw = 192 if mode == 0 else 64
    sw = 64 if mode == 0 else 192
    gdtype = _F32 if mode == 0 else jnp.bfloat16
    sdtype = _F32 if mode == 0 else jnp.bfloat16
    nch = NCH
    mesh = plsc.VectorSubcoreMesh(core_axis_name="c", subcore_axis_name="s",
                                  num_cores=2, num_subcores=16)

    scratch = [
        pltpu.VMEM((5, CH), jnp.int32),
        pltpu.VMEM((5, CH), jnp.int32),
        pltpu.VMEM((5, CH), jnp.int32),
        pltpu.VMEM((5, CH), jnp.int32),
        pltpu.VMEM((CH, gw), gdtype),
        pltpu.VMEM((CH, gw), gdtype),
        pltpu.VMEM((CH, sw), sdtype),
        pltpu.VMEM((CH, sw), sdtype),
        pltpu.VMEM_SHARED((NPAD, sw), sdtype),
    ] + [pltpu.SemaphoreType.DMA] * 8
    if mode != 0:
        scratch = scratch + [pltpu.VMEM((CH, 96), jnp.bfloat16)] * 4 \
            + [pltpu.SemaphoreType.DMA] * 4

    def scgs_body(*refs):
        if mode == 0:
            (y_hbm, ed_hbm, z_hbm, acc_hbm,
             ed0, ed1, ed2, ed3, rw0, rw1, mg0, mg1, acc_sh,
             sg0, sg1, se0, se1, se2, se3, ss0, ss1) = refs
            cbs = scs = None
        else:
            (y_hbm, ed_hbm, cb_hbm, z_hbm, acc_hbm,
             ed0, ed1, ed2, ed3, rw0, rw1, mg0, mg1, acc_sh,
             sg0, sg1, se0, se1, se2, se3, ss0, ss1,
             cb0, cb1, cb2, cb3, sc0, sc1, sc2, sc3) = refs
            cbs = (cb0, cb1, cb2, cb3)
            scs = (sc0, sc1, sc2, sc3)
        cid = lax.axis_index("c")
        sid = lax.axis_index("s")
        eds = (ed0, ed1, ed2, ed3)
        rows = (rw0, rw1)
        msgs = (mg0, mg1)
        sgs = (sg0, sg1)
        ses = (se0, se1, se2, se3)
        sss = (ss0, ss1)

        pltpu.sync_copy(z_hbm.at[pl.ds(sid * ZR, ZR)],
                        acc_sh.at[pl.ds(sid * ZR, ZR)])
        plsc.subcore_barrier()

        if mode == 0:
            cpt0 = _CPT_CORE0
            cpt1 = (EPAD // CH - 16 * cpt0) // 16
            nch_w = jnp.where(cid == 0, cpt0, cpt1)
            ci0 = jnp.where(cid == 0, sid * cpt0, 16 * cpt0 + sid * cpt1)
        else:
            nch_w = nch
            ci0 = (sid * 2 + cid) * nch

        def ed_src(c):
            return ed_hbm.at[pl.ds((ci0 + c) * 5, 5)]

        def cb_src(c):
            return cb_hbm.at[pl.ds((ci0 + c) * CH, CH)]

        pltpu.sync_copy(ed_src(0), ed0)
        pltpu.async_copy(y_hbm.at[ed0.at[0]], rw0, sg0)
        pltpu.async_copy(ed_src(1), ed1, se1)
        if mode != 0:
            pltpu.async_copy(cb_src(0), cbs[0], scs[0])
            pltpu.async_copy(cb_src(1), cbs[1], scs[1])

        def compute_chunk(ed_v, rows_v, msg_v, cb_v):
            if mode == 0:
                def group_body(g, carry):
                    c0g = lax.bitcast_convert_type(
                        ed_v[2, pl.ds(g * 16, 16)], _F32)
                    c1g = lax.bitcast_convert_type(
                        ed_v[3, pl.ds(g * 16, 16)], _F32)
                    c2g = lax.bitcast_convert_type(
                        ed_v[4, pl.ds(g * 16, 16)], _F32)
                    for t in range(16):
                        e = g * 16 + t
                        c0 = c0g[t]
                        c1 = c1g[t]
                        c2 = c2g[t]
                        for j in range(4):
                            msg_v[e, pl.ds(j * 16, 16)] = (
                                c0 * rows_v[e, pl.ds(j * 16, 16)]
                                + c1 * rows_v[e, pl.ds(64 + j * 16, 16)]
                                + c2 * rows_v[e, pl.ds(128 + j * 16, 16)])
                    return carry

                lax.fori_loop(0, CH // 16, group_body, 0)
            else:
                def edge_body(e, carry):
                    hv = [rows_v[e, pl.ds(q * 32, 32)] for q in range(2)]
                    for li in range(3):
                        cl = cb_v[e, pl.ds(li * 32, 32)]
                        for j in range(2):
                            msg_v[e, pl.ds(li * 64 + j * 32, 32)] = cl * hv[j]
                    return carry

                lax.fori_loop(0, CH, edge_body, 0)

        def step(s, carry):
            for b in range(4):
                c = s * 4 + b
                gb = b % 2
                nb = (b + 1) % 2
                @pl.when(c + 1 < nch_w)
                def _():
                    pltpu.make_async_copy(
                        ed_src(c + 1), eds[(b + 1) % 4],
                        ses[(b + 1) % 4]).wait()
                    pltpu.async_copy(
                        y_hbm.at[eds[(b + 1) % 4].at[0]], rows[nb], sgs[nb])

                pltpu.make_async_copy(
                    y_hbm.at[eds[b].at[0]], rows[gb], sgs[gb]).wait()

                @pl.when(c >= 2)
                def _():
                    pltpu.make_async_copy(
                        msgs[gb], acc_sh.at[eds[(b + 2) % 4].at[1]],
                        sss[gb]).wait()

                @pl.when(c + 2 < nch_w)
                def _():
                    pltpu.async_copy(ed_src(c + 2), eds[(b + 2) % 4],
                                     ses[(b + 2) % 4])
                    if mode != 0:
                        pltpu.async_copy(cb_src(c + 2), cbs[(b + 2) % 4],
                                         scs[(b + 2) % 4])

                if mode != 0:
                    pltpu.make_async_copy(cb_src(c), cbs[b], scs[b]).wait()
                compute_chunk(eds[b], rows[gb], msgs[gb],
                              cbs[b] if mode != 0 else None)
                pltpu.async_copy(msgs[gb], acc_sh.at[eds[b].at[1]],
                                 sss[gb], add=True)
            return carry

        lax.fori_loop(0, nch_w // 4, step, 0)
        pltpu.make_async_copy(msgs[0], acc_sh.at[eds[2].at[1]], sss[0]).wait()
        pltpu.make_async_copy(msgs[1], acc_sh.at[eds[3].at[1]], sss[1]).wait()
        plsc.subcore_barrier()
        pltpu.sync_copy(acc_sh.at[pl.ds(sid * ZR, ZR)],
                        acc_hbm.at[cid, pl.ds(sid * ZR, ZR)])

    return pl.kernel(
        scgs_body,
        out_type=jax.ShapeDtypeStruct((2, NPAD, sw), sdtype),
        mesh=mesh,
        compiler_params=pltpu.CompilerParams(use_tc_tiling_on_sc=False),
        scratch_types=scratch,
    )


@functools.lru_cache(maxsize=None)
def _get_scgs(mode):
    return _make_scgs(mode)


def _scgs_wide(*args):
    return _get_scgs(0)(*args)


def _scgs_outer(*args):
    return _get_scgs(1)(*args)


def _pack_ed(src_r, dst_r, c0, c1, c2):
    cb = [lax.bitcast_convert_type(c.reshape(EPAD // CH, CH), jnp.int32)
          for c in (c0, c1, c2)]
    return jnp.stack([src_r, dst_r, *cb], axis=1).reshape(EPAD // CH * 5, CH)


def kernel(x, edge_index, edge_vec, Wl0, Wself0, fc1_0, fc2_0,
           Wl1, Wself1, fc1_1, fc2_1, Wl2, Wself2, fc1_2, fc2_2):
    pad_e = EPAD - N_EDGES
    src = jnp.concatenate([edge_index[0], jnp.zeros((pad_e,), jnp.int32)])
    dst = jnp.concatenate([edge_index[1], jnp.zeros((pad_e,), jnp.int32)])
    evt = jnp.pad(edge_vec.T, ((0, 0), (0, pad_e)))

    wcat0 = jnp.concatenate([Wl0[0], Wl0[1], Wl0[2], Wself0], axis=1)
    wcat1 = jnp.concatenate([Wl1[0], Wl1[1], Wl1[2], Wself1], axis=1)
    w2stack = jnp.concatenate([Wl2[0], Wl2[1], Wl2[2]], axis=0)
    zeros64 = jnp.zeros((NPAD, 64), _F32)
    zeros192b = jnp.zeros((NPAD, 192), jnp.bfloat16)

    c_all = _coef_call(evt,
                       (fc1_0.T, fc1_1.T, fc1_2.T),
                       (fc2_0.T, fc2_1.T, fc2_2.T))

    src_r = src.reshape(EPAD // CH, CH)
    dst_r = dst.reshape(EPAD // CH, CH)
    ed_l0 = _pack_ed(src_r, dst_r, c_all[0], c_all[1], c_all[2])
    ed_l1 = _pack_ed(src_r, dst_r, c_all[3], c_all[4], c_all[5])
    ed_l2 = _pack_ed(src_r, dst_r, c_all[6], c_all[7], c_all[8])
    cb_l2 = jnp.concatenate(
        [jnp.broadcast_to(c_all[6 + l].astype(jnp.bfloat16)[:, None],
                          (EPAD, 32)) for l in range(3)], axis=1)

    y0, s0 = _mm0_call(x, wcat0)
    acc0 = _scgs_wide(y0, ed_l0, zeros64)
    y1, s1 = _mmc_call(s0, acc0, wcat1)
    acc1 = _scgs_wide(y1, ed_l1, zeros64)
    h2, h2b = _comb2_call(s1, acc1)
    a2 = _scgs_outer(h2b, ed_l2, cb_l2, zeros192b)
    return _final_call(h2, a2, Wself2, w2stack)


# speedup vs baseline: 3.0211x; 1.0109x over previous
import functools

import jax
import jax.numpy as jnp
import numpy as np
from jax import lax
from jax.experimental import pallas as pl
from jax.experimental.pallas import tpu as pltpu
from jax.experimental.pallas import tpu_sc as plsc

N_NODES = 10000
N_EDGES = 160000
NPAD = 10240
EPAD = 163840
CH = 128
EPW = EPAD // 32
NCH = EPW // CH
ZR = NPAD // 16

_F32 = jnp.float32
_CPT_CORE0 = 52

_STEP = 3.5 / 11.0
_EMB_SCALE = 1.14136 * float(np.exp(2.0)) * float(np.sqrt(10.0))


def _coef_body(ev_ref, f1t0, f2t0, f1t1, f2t1, f1t2, f2t2, c_ref):
    ev = ev_ref[...]
    x = ev[0:1, :]
    y = ev[1:2, :]
    z = ev[2:3, :]
    r2 = x * x + y * y + z * z
    r = jnp.sqrt(r2 + 1e-12)
    inv = 1.0 / r
    ux, uy, uz = x * inv, y * inv, z * inv
    s1 = 1.7320508075688772 * (ux + uy + uz)
    s2 = (3.872983346207417 * (ux * uy + uy * uz + ux * uz)
          + 1.118033988749895 * (3.0 * uz * uz - 1.0)
          + 1.9364916731037085 * (ux * ux - uy * uy))
    s_all = jnp.concatenate([jnp.ones_like(s1), s1, s2], axis=0)

    vals = (lax.broadcasted_iota(jnp.int32, (10, 1), 0).astype(_F32) + 1.0) * _STEP
    diff = (r - vals) * (1.0 / _STEP)

    def _sus(t):
        return jnp.where(t > 0.0, jnp.exp(-1.0 / jnp.clip(t, 1e-8, None)), 0.0)

    emb = _EMB_SCALE * _sus(diff + 1.0) * _sus(1.0 - diff)

    rows = []
    for f1t, f2t in ((f1t0, f2t0), (f1t1, f2t1), (f1t2, f2t2)):
        hid = jnp.dot(f1t[...], emb, preferred_element_type=_F32)
        hid = hid * (0.5 + 0.5 * jnp.tanh(0.5 * hid))
        w = jnp.dot(f2t[...], hid, preferred_element_type=_F32)
        rows.append(w * s_all)
    c = jnp.concatenate(rows, axis=0)
    c_ref[...] = jnp.concatenate([c, jnp.zeros_like(c[0:7, :])], axis=0)


def _coef_call(evt, f1ts, f2ts):
    be = 8192
    grid = (EPAD // be,)
    full = pl.BlockSpec((None, None), lambda i: (0, 0))
    specs = [pl.BlockSpec((3, be), lambda i: (0, i))]
    for _ in range(3):
        specs.append(pl.BlockSpec((100, 10), lambda i: (0, 0)))
        specs.append(pl.BlockSpec((3, 100), lambda i: (0, 0)))
    args = [evt]
    for f1t, f2t in zip(f1ts, f2ts):
        args += [f1t, f2t]
    return pl.pallas_call(
        _coef_body,
        grid=grid,
        in_specs=specs,
        out_specs=pl.BlockSpec((16, be), lambda i: (0, i)),
        out_shape=jax.ShapeDtypeStruct((16, EPAD), _F32),
    )(*args)


def _mm0_body(x_ref, w_ref, y_ref, s_ref):
    t = jnp.dot(x_ref[...], w_ref[...], preferred_element_type=_F32)
    y_ref[...] = t[:, :192]
    s_ref[...] = t[:, 192:]


def _mm0_call(x, wcat):
    br = 2000
    return pl.pallas_call(
        _mm0_body,
        grid=(N_NODES // br,),
        in_specs=[pl.BlockSpec((br, 256), lambda i: (i, 0)),
                  pl.BlockSpec((256, 256), lambda i: (0, 0))],
        out_specs=[pl.BlockSpec((br, 192), lambda i: (i, 0)),
                   pl.BlockSpec((br, 64), lambda i: (i, 0))],
        out_shape=[jax.ShapeDtypeStruct((N_NODES, 192), _F32),
                   jax.ShapeDtypeStruct((N_NODES, 64), _F32)],
    )(x, wcat)


def _mmc_body(s_ref, acc_ref, w_ref, y_ref, sn_ref):
    pre = s_ref[...] + 0.25 * (acc_ref[0] + acc_ref[1])
    h = pre * (0.5 + 0.5 * jnp.tanh(0.5 * pre))
    t = jnp.dot(h, w_ref[...], preferred_element_type=_F32)
    y_ref[...] = t[:, :192]
    sn_ref[...] = t[:, 192:]


def _mmc_call(s_prev, acc, wcat):
    br = 2000
    return pl.pallas_call(
        _mmc_body,
        grid=(N_NODES // br,),
        in_specs=[pl.BlockSpec((br, 64), lambda i: (i, 0)),
                  pl.BlockSpec((2, br, 64), lambda i: (0, i, 0)),
                  pl.BlockSpec((64, 256), lambda i: (0, 0))],
        out_specs=[pl.BlockSpec((br, 192), lambda i: (i, 0)),
                   pl.BlockSpec((br, 64), lambda i: (i, 0))],
        out_shape=[jax.ShapeDtypeStruct((N_NODES, 192), _F32),
                   jax.ShapeDtypeStruct((N_NODES, 64), _F32)],
    )(s_prev, acc, wcat)


def _comb2_body(s_ref, acc_ref, h_ref, hb_ref):
    pre = s_ref[...] + 0.25 * (acc_ref[0] + acc_ref[1])
    h = pre * (0.5 + 0.5 * jnp.tanh(0.5 * pre))
    h_ref[...] = h
    hb_ref[...] = h.astype(jnp.bfloat16)


def _comb2_call(s_prev, acc):
    br = 2000
    return pl.pallas_call(
        _comb2_body,
        grid=(N_NODES // br,),
        in_specs=[pl.BlockSpec((br, 64), lambda i: (i, 0)),
                  pl.BlockSpec((2, br, 64), lambda i: (0, i, 0))],
        out_specs=[pl.BlockSpec((br, 64), lambda i: (i, 0)),
                   pl.BlockSpec((br, 64), lambda i: (i, 0))],
        out_shape=[jax.ShapeDtypeStruct((N_NODES, 64), _F32),
                   jax.ShapeDtypeStruct((N_NODES, 64), jnp.bfloat16)],
    )(s_prev, acc)


def _final_body(h_ref, a_ref, ws_ref, w2_ref, o_ref):
    a = a_ref[0].astype(_F32) + a_ref[1].astype(_F32)
    o_ref[...] = (jnp.dot(h_ref[...], ws_ref[...], preferred_element_type=_F32)
                  + 0.25 * jnp.dot(a, w2_ref[...], preferred_element_type=_F32))


def _final_call(h2, acc, wself2, w2stack):
    br = 2000
    return pl.pallas_call(
        _final_body,
        grid=(N_NODES // br,),
        in_specs=[pl.BlockSpec((br, 64), lambda i: (i, 0)),
                  pl.BlockSpec((2, br, 192), lambda i: (0, i, 0)),
                  pl.BlockSpec((64, 256), lambda i: (0, 0)),
                  pl.BlockSpec((192, 256), lambda i: (0, 0))],
        out_specs=pl.BlockSpec((br, 256), lambda i: (i, 0)),
        out_shape=jax.ShapeDtypeStruct((N_NODES, 256), _F32),
    )(h2, acc, wself2, w2stack)


def _make_scgs(mode):
    gw = 192 if mode == 0 else 64
    sw = 64 if mode == 0 else 192
    gdtype = _F32 if mode == 0 else jnp.bfloat16
    sdtype = _F32 if mode == 0 else jnp.bfloat16
    nch = NCH
    mesh = plsc.VectorSubcoreMesh(core_axis_name="c", subcore_axis_name="s",
                                  num_cores=2, num_subcores=16)

    scratch = [
        pltpu.VMEM((5, CH), jnp.int32),
        pltpu.VMEM((5, CH), jnp.int32),
        pltpu.VMEM((5, CH), jnp.int32),
        pltpu.VMEM((5, CH), jnp.int32),
        pltpu.VMEM((CH, gw), gdtype),
        pltpu.VMEM((CH, gw), gdtype),
        pltpu.VMEM((CH, sw), sdtype),
        pltpu.VMEM((CH, sw), sdtype),
        pltpu.VMEM_SHARED((NPAD, sw), sdtype),
    ] + [pltpu.SemaphoreType.DMA] * 8
    if mode != 0:
        scratch = scratch + [pltpu.VMEM((CH, 96), jnp.bfloat16)] * 4 \
            + [pltpu.SemaphoreType.DMA] * 4

    def scgs_body(*refs):
        if mode == 0:
            (y_hbm, ed_hbm, z_hbm, acc_hbm,
             ed0, ed1, ed2, ed3, rw0, rw1, mg0, mg1, acc_sh,
             sg0, sg1, se0, se1, se2, se3, ss0, ss1) = refs
            cbs = scs = None
        else:
            (y_hbm, ed_hbm, cb_hbm, acc_hbm,
             ed0, ed1, ed2, ed3, rw0, rw1, mg0, mg1, acc_sh,
             sg0, sg1, se0, se1, se2, se3, ss0, ss1,
             cb0, cb1, cb2, cb3, sc0, sc1, sc2, sc3) = refs
            cbs = (cb0, cb1, cb2, cb3)
            scs = (sc0, sc1, sc2, sc3)
        cid = lax.axis_index("c")
        sid = lax.axis_index("s")
        eds = (ed0, ed1, ed2, ed3)
        rows = (rw0, rw1)
        msgs = (mg0, mg1)
        sgs = (sg0, sg1)
        ses = (se0, se1, se2, se3)
        sss = (ss0, ss1)

        if mode == 0:
            pltpu.sync_copy(z_hbm.at[pl.ds(sid * ZR, ZR)],
                            acc_sh.at[pl.ds(sid * ZR, ZR)])
        else:
            zv = jnp.zeros((32,), jnp.bfloat16)

            def zrow(r, carry):
                for k in range(sw // 32):
                    mg0[r, pl.ds(k * 32, 32)] = zv
                return carry

            lax.fori_loop(0, CH, zrow, 0)
            for i in range(ZR // CH):
                pltpu.sync_copy(mg0, acc_sh.at[pl.ds(sid * ZR + i * CH, CH)])
        plsc.subcore_barrier()

        if mode == 0:
            cpt0 = _CPT_CORE0
            cpt1 = (EPAD // CH - 16 * cpt0) // 16
            nch_w = jnp.where(cid == 0, cpt0, cpt1)
            ci0 = jnp.where(cid == 0, sid * cpt0, 16 * cpt0 + sid * cpt1)
        else:
            nch_w = nch
            ci0 = (sid * 2 + cid) * nch

        def ed_src(c):
            return ed_hbm.at[pl.ds((ci0 + c) * 5, 5)]

        def cb_src(c):
            return cb_hbm.at[pl.ds((ci0 + c) * CH, CH)]

        pltpu.sync_copy(ed_src(0), ed0)
        pltpu.async_copy(y_hbm.at[ed0.at[0]], rw0, sg0)
        pltpu.async_copy(ed_src(1), ed1, se1)
        if mode != 0:
            pltpu.async_copy(cb_src(0), cbs[0], scs[0])
            pltpu.async_copy(cb_src(1), cbs[1], scs[1])

        def compute_chunk(ed_v, rows_v, msg_v, cb_v):
            if mode == 0:
                def group_body(g, carry):
                    c0g = lax.bitcast_convert_type(
                        ed_v[2, pl.ds(g * 16, 16)], _F32)
                    c1g = lax.bitcast_convert_type(
                        ed_v[3, pl.ds(g * 16, 16)], _F32)
                    c2g = lax.bitcast_convert_type(
                        ed_v[4, pl.ds(g * 16, 16)], _F32)
                    for t in range(16):
                        e = g * 16 + t
                        c0 = c0g[t]
                        c1 = c1g[t]
                        c2 = c2g[t]
                        for j in range(4):
                            msg_v[e, pl.ds(j * 16, 16)] = (
                                c0 * rows_v[e, pl.ds(j * 16, 16)]
                                + c1 * rows_v[e, pl.ds(64 + j * 16, 16)]
                                + c2 * rows_v[e, pl.ds(128 + j * 16, 16)])
                    return carry

                lax.fori_loop(0, CH // 16, group_body, 0)
            else:
                def edge_body(e, carry):
                    hv = [rows_v[e, pl.ds(q * 32, 32)] for q in range(2)]
                    for li in range(3):
                        cl = cb_v[e, pl.ds(li * 32, 32)]
                        for j in range(2):
                            msg_v[e, pl.ds(li * 64 + j * 32, 32)] = cl * hv[j]
                    return carry

                lax.fori_loop(0, CH, edge_body, 0)

        def step(s, carry):
            for b in range(4):
                c = s * 4 + b
                gb = b % 2
                nb = (b + 1) % 2
                @pl.when(c + 1 < nch_w)
                def _():
                    pltpu.make_async_copy(
                        ed_src(c + 1), eds[(b + 1) % 4],
                        ses[(b + 1) % 4]).wait()
                    pltpu.async_copy(
                        y_hbm.at[eds[(b + 1) % 4].at[0]], rows[nb], sgs[nb])

                pltpu.make_async_copy(
                    y_hbm.at[eds[b].at[0]], rows[gb], sgs[gb]).wait()

                @pl.when(c >= 2)
                def _():
                    pltpu.make_async_copy(
                        msgs[gb], acc_sh.at[eds[(b + 2) % 4].at[1]],
                        sss[gb]).wait()

                @pl.when(c + 2 < nch_w)
                def _():
                    pltpu.async_copy(ed_src(c + 2), eds[(b + 2) % 4],
                                     ses[(b + 2) % 4])
                    if mode != 0:
                        pltpu.async_copy(cb_src(c + 2), cbs[(b + 2) % 4],
                                         scs[(b + 2) % 4])

                if mode != 0:
                    pltpu.make_async_copy(cb_src(c), cbs[b], scs[b]).wait()
                compute_chunk(eds[b], rows[gb], msgs[gb],
                              cbs[b] if mode != 0 else None)
                pltpu.async_copy(msgs[gb], acc_sh.at[eds[b].at[1]],
                                 sss[gb], add=True)
            return carry

        lax.fori_loop(0, nch_w // 4, step, 0)
        pltpu.make_async_copy(msgs[0], acc_sh.at[eds[2].at[1]], sss[0]).wait()
        pltpu.make_async_copy(msgs[1], acc_sh.at[eds[3].at[1]], sss[1]).wait()
        plsc.subcore_barrier()
        pltpu.sync_copy(acc_sh.at[pl.ds(sid * ZR, ZR)],
                        acc_hbm.at[cid, pl.ds(sid * ZR, ZR)])

    return pl.kernel(
        scgs_body,
        out_type=jax.ShapeDtypeStruct((2, NPAD, sw), sdtype),
        mesh=mesh,
        compiler_params=pltpu.CompilerParams(use_tc_tiling_on_sc=False),
        scratch_types=scratch,
    )


@functools.lru_cache(maxsize=None)
def _get_scgs(mode):
    return _make_scgs(mode)


def _scgs_wide(*args):
    return _get_scgs(0)(*args)


def _scgs_outer(*args):
    return _get_scgs(1)(*args)


def _pack_ed(src_r, dst_r, c0, c1, c2):
    cb = [lax.bitcast_convert_type(c.reshape(EPAD // CH, CH), jnp.int32)
          for c in (c0, c1, c2)]
    return jnp.stack([src_r, dst_r, *cb], axis=1).reshape(EPAD // CH * 5, CH)


def kernel(x, edge_index, edge_vec, Wl0, Wself0, fc1_0, fc2_0,
           Wl1, Wself1, fc1_1, fc2_1, Wl2, Wself2, fc1_2, fc2_2):
    pad_e = EPAD - N_EDGES
    src = jnp.concatenate([edge_index[0], jnp.zeros((pad_e,), jnp.int32)])
    dst = jnp.concatenate([edge_index[1], jnp.zeros((pad_e,), jnp.int32)])
    evt = jnp.pad(edge_vec.T, ((0, 0), (0, pad_e)))

    wcat0 = jnp.concatenate([Wl0[0], Wl0[1], Wl0[2], Wself0], axis=1)
    wcat1 = jnp.concatenate([Wl1[0], Wl1[1], Wl1[2], Wself1], axis=1)
    w2stack = jnp.concatenate([Wl2[0], Wl2[1], Wl2[2]], axis=0)
    zeros64 = jnp.zeros((NPAD, 64), _F32)

    c_all = _coef_call(evt,
                       (fc1_0.T, fc1_1.T, fc1_2.T),
                       (fc2_0.T, fc2_1.T, fc2_2.T))

    src_r = src.reshape(EPAD // CH, CH)
    dst_r = dst.reshape(EPAD // CH, CH)
    ed_l0 = _pack_ed(src_r, dst_r, c_all[0], c_all[1], c_all[2])
    ed_l1 = _pack_ed(src_r, dst_r, c_all[3], c_all[4], c_all[5])
    ed_l2 = _pack_ed(src_r, dst_r, c_all[6], c_all[7], c_all[8])
    cb_l2 = jnp.concatenate(
        [jnp.broadcast_to(c_all[6 + l].astype(jnp.bfloat16)[:, None],
                          (EPAD, 32)) for l in range(3)], axis=1)

    y0, s0 = _mm0_call(x, wcat0)
    acc0 = _scgs_wide(y0, ed_l0, zeros64)
    y1, s1 = _mmc_call(s0, acc0, wcat1)
    acc1 = _scgs_wide(y1, ed_l1, zeros64)
    h2, h2b = _comb2_call(s1, acc1)
    a2 = _scgs_outer(h2b, ed_l2, cb_l2)
    return _final_call(h2, a2, Wself2, w2stack)
